# baseline (speedup 1.0000x reference)
import functools

import jax
import jax.numpy as jnp
from jax.experimental import pallas as pl


def _bn_cols(x, g, b):
    m = jnp.mean(x, axis=0, keepdims=True)
    v = jnp.mean((x - m) ** 2, axis=0, keepdims=True)
    return (x - m) * jax.lax.rsqrt(v + 1e-5) * g + b


def _dot(a, b):
    return jnp.dot(a, b, preferred_element_type=jnp.float32)


def _pre_body(feat_ref, g_ref, b_ref, wih_t_ref, bih_ref, wself_t_ref,
              fb_ref, a_ref, s_ref):
    fb = _bn_cols(feat_ref[...], g_ref[...], b_ref[...])
    fb_ref[...] = fb
    a_ref[...] = _dot(fb, wih_t_ref[...]) + bih_ref[...]
    s_ref[...] = _dot(fb, wself_t_ref[...])


def _gru_step_body(gi_ref, h_ref, whh_t_ref, bhh_ref, mask_ref, out_ref):
    d = h_ref.shape[1]
    h = h_ref[...]
    gi = gi_ref[...]
    gh = _dot(h, whh_t_ref[...]) + bhh_ref[...]
    ir, iz, inew = gi[:, :d], gi[:, d:2 * d], gi[:, 2 * d:]
    hr, hz, hnn = gh[:, :d], gh[:, d:2 * d], gh[:, 2 * d:]
    r = jax.nn.sigmoid(ir + hr)
    z = jax.nn.sigmoid(iz + hz)
    cand = jnp.tanh(inew + r * hnn)
    hn = (1.0 - z) * cand + z * h
    m = mask_ref[...]
    out_ref[...] = m * hn + (1.0 - m) * h


def _post_eopa_body(s_ref, h_ref, wneigh_t_ref, a0_ref, feat_ref,
                    g1_ref, b1_ref, wq_t_ref, qb_ref, wk_t_ref, wv_t_ref,
                    feat1_ref, q_ref, k_ref, v_ref):
    out0 = s_ref[...] + _dot(h_ref[...], wneigh_t_ref[...])
    a0 = a0_ref[...]
    out0 = jnp.where(out0 > 0, out0, a0 * out0)
    feat1 = jnp.concatenate([out0, feat_ref[...]], axis=1)
    feat1_ref[...] = feat1
    fb1 = _bn_cols(feat1, g1_ref[...], b1_ref[...])
    q_ref[...] = _dot(fb1, wq_t_ref[...]) + qb_ref[...]
    k_ref[...] = _dot(fb1, wk_t_ref[...])
    v_ref[...] = _dot(fb1, wv_t_ref[...])


def _feat2_body(rst_ref, a1_ref, feat1_ref, feat2_ref):
    rst = rst_ref[...]
    a1 = a1_ref[...]
    out1 = jnp.where(rst > 0, rst, a1 * rst)
    feat2_ref[...] = jnp.concatenate([out1, feat1_ref[...]], axis=1)


def _bn2_body(feat2_ref, g2_ref, b2_ref, wu_t_ref, fb2_ref, fu_ref):
    fb2 = _bn_cols(feat2_ref[...], g2_ref[...], b2_ref[...])
    fb2_ref[...] = fb2
    fu_ref[...] = _dot(fb2, wu_t_ref[...])


def _final_body(srg_ref, wout_t_ref, a2_ref, feat2l_ref, wrf_t_ref,
                rfb_ref, wsr_t_ref, dec_ref, logits_ref):
    srg = _dot(srg_ref[...], wout_t_ref[...])
    a2 = a2_ref[...]
    srg = jnp.where(srg > 0, srg, a2 * srg)
    sr = jnp.concatenate([feat2l_ref[...], srg], axis=1)
    dec_ref[...] = _dot(sr, wrf_t_ref[...]) + rfb_ref[...]
    logits_ref[...] = _dot(sr, wsr_t_ref[...])


def _call(body, out_shapes, *args):
    return pl.pallas_call(
        body,
        out_shape=out_shapes,
    )(*args)


def _seg_softmax(e, seg, num):
    m = jax.ops.segment_max(e, seg, num_segments=num)
    m = jnp.where(jnp.isfinite(m), m, 0.0)
    ex = jnp.exp(e - m[seg])
    den = jax.ops.segment_sum(ex, seg, num_segments=num)
    return ex / den[seg]


def kernel(iid, edge_index_mg, edge_index_sg, segment_ids, last_nodes,
           emb_table, bn0_gamma, bn0_beta, gru_Wih, gru_Whh, gru_bih, gru_bhh,
           fc_self_W, fc_neigh_W, prelu0_a, bn1_gamma, bn1_beta,
           fc_q_W, fc_q_b, fc_k_W, fc_v_W, fc_e1_W, prelu1_a,
           bn2_gamma, bn2_beta, fc_u_W, fc_v2_W, fc_v2_b, fc_e2_W, fc_out_W,
           prelu2_a, fc_RF_W, fc_RF_b, fc_sr_W):
    n = iid.shape[0]
    d = emb_table.shape[1]
    b = last_nodes.shape[0]
    f32 = jnp.float32

    def row(x):
        return x.reshape(1, -1).astype(f32)

    feat = emb_table[iid]

    src, dst = edge_index_mg[0], edge_index_mg[1]
    order = jnp.argsort(dst)
    src_s = src[order]
    counts = jnp.bincount(dst, length=n)
    offsets = jnp.cumsum(counts) - counts
    maxdeg = counts.max()

    fb0, A, S = _call(
        _pre_body,
        (jax.ShapeDtypeStruct((n, d), f32),
         jax.ShapeDtypeStruct((n, 3 * d), f32),
         jax.ShapeDtypeStruct((n, d), f32)),
        feat, row(bn0_gamma), row(bn0_beta),
        gru_Wih.T.astype(f32), row(gru_bih), fc_self_W.T.astype(f32))

    whh_t = gru_Whh.T.astype(f32)
    bhh_r = row(gru_bhh)

    def gru_body(t, h):
        mk = t < counts
        idx = jnp.where(mk, offsets + t, 0)
        gi = A[src_s[idx]]
        maskf = mk.astype(f32)[:, None]
        return _call(
            _gru_step_body,
            jax.ShapeDtypeStruct((n, d), f32),
            gi, h, whh_t, bhh_r, maskf)

    h0 = jnp.zeros((n, d), f32)
    neigh = jax.lax.fori_loop(0, maxdeg, gru_body, h0)

    feat1, q, k, v = _call(
        _post_eopa_body,
        (jax.ShapeDtypeStruct((n, 2 * d), f32),
         jax.ShapeDtypeStruct((n, d), f32),
         jax.ShapeDtypeStruct((n, d), f32),
         jax.ShapeDtypeStruct((n, d), f32)),
        S, neigh, fc_neigh_W.T.astype(f32), row(prelu0_a), feat,
        row(bn1_gamma), row(bn1_beta),
        fc_q_W.T.astype(f32), row(fc_q_b),
        fc_k_W.T.astype(f32), fc_v_W.T.astype(f32))

    src2, dst2 = edge_index_sg[0], edge_index_sg[1]
    e = jax.nn.sigmoid(q[src2] + k[dst2]) @ fc_e1_W.T
    a_att = _seg_softmax(e, dst2, n)
    rst = jax.ops.segment_sum(v[src2] * a_att, dst2, num_segments=n)

    feat2 = _call(
        _feat2_body,
        jax.ShapeDtypeStruct((n, 3 * d), f32),
        rst, row(prelu1_a), feat1)
    fb2, fu = _call(
        _bn2_body,
        (jax.ShapeDtypeStruct((n, 3 * d), f32),
         jax.ShapeDtypeStruct((n, d), f32)),
        feat2, row(bn2_gamma), row(bn2_beta), fc_u_W.T.astype(f32))

    fv = fb2[last_nodes] @ fc_v2_W.T + fc_v2_b
    e2 = jax.nn.sigmoid(fu + fv[segment_ids]) @ fc_e2_W.T
    alpha = _seg_softmax(e2, segment_ids, b)
    sr_g = jax.ops.segment_sum(fb2 * alpha, segment_ids, num_segments=b)

    dec, logits = _call(
        _final_body,
        (jax.ShapeDtypeStruct((b, fc_RF_W.shape[0]), f32),
         jax.ShapeDtypeStruct((b, fc_sr_W.shape[0]), f32)),
        sr_g, fc_out_W.T.astype(f32), row(prelu2_a), feat2[last_nodes],
        fc_RF_W.T.astype(f32), row(fc_RF_b), fc_sr_W.T.astype(f32))

    return (dec, logits)


# speedup vs baseline: 3.5278x; 3.5278x over previous
import functools

import jax
import jax.numpy as jnp
from jax.experimental import pallas as pl
from jax.experimental.pallas import tpu as pltpu


def _bn_cols(x, g, b):
    m = jnp.mean(x, axis=0, keepdims=True)
    v = jnp.mean((x - m) ** 2, axis=0, keepdims=True)
    return (x - m) * jax.lax.rsqrt(v + 1e-5) * g + b


def _dot(a, b):
    return jnp.dot(a, b, preferred_element_type=jnp.float32)


def _pre_body(feat_ref, g_ref, b_ref, wself_t_ref, fb_ref, s_ref):
    fb = _bn_cols(feat_ref[...], g_ref[...], b_ref[...])
    fb_ref[...] = fb
    s_ref[...] = _dot(fb, wself_t_ref[...])


_GRU_BLK = 1024


def _gru_mega_body(nsteps_ref, countsf_ref, wih_t_ref, bih_ref, whh_t_ref,
                   bhh_ref, xs_ref, out_ref, xbuf_ref, sem_ref):
    blk = _GRU_BLK
    d = out_ref.shape[1]
    out_ref[...] = jnp.zeros(out_ref.shape, out_ref.dtype)
    nsteps = nsteps_ref[0]
    cf = countsf_ref[...]
    wih = wih_t_ref[...]
    whh = whh_t_ref[...]
    bih = bih_ref[...]
    bhh = bhh_ref[...]

    def dma(slot, start):
        return pltpu.make_async_copy(
            xs_ref.at[pl.ds(start, blk)], xbuf_ref.at[slot], sem_ref.at[slot])

    def step(t, R):
        tf = t.astype(jnp.float32)
        K = jnp.sum((cf > tf).astype(jnp.float32)).astype(jnp.int32)
        nblk = (K + blk - 1) // blk
        dma(0, R).start()

        def body(b, _):
            base = b * blk
            slot = jax.lax.rem(b, 2)

            @pl.when(b + 1 < nblk)
            def _():
                dma(jax.lax.rem(b + 1, 2), R + base + blk).start()

            dma(slot, R + base).wait()
            x = xbuf_ref[slot]
            gi = _dot(x, wih) + bih
            h = out_ref[pl.ds(base, blk), :]
            gh = _dot(h, whh) + bhh
            ir, iz, inew = gi[:, :d], gi[:, d:2 * d], gi[:, 2 * d:]
            hr, hz, hnn = gh[:, :d], gh[:, d:2 * d], gh[:, 2 * d:]
            r = jax.nn.sigmoid(ir + hr)
            z = jax.nn.sigmoid(iz + hz)
            cand = jnp.tanh(inew + r * hnn)
            hn = (1.0 - z) * cand + z * h
            rowid = base + jax.lax.broadcasted_iota(jnp.int32, (blk, 1), 0)
            out_ref[pl.ds(base, blk), :] = jnp.where(rowid < K, hn, h)
            return 0

        jax.lax.fori_loop(0, nblk, body, 0)
        return R + K

    jax.lax.fori_loop(0, nsteps, step, jnp.int32(0))


def _post_eopa_body(s_ref, h_ref, wneigh_t_ref, a0_ref, feat_ref,
                    g1_ref, b1_ref, wq_t_ref, qb_ref, wk_t_ref, wv_t_ref,
                    feat1_ref, q_ref, k_ref, v_ref):
    out0 = s_ref[...] + _dot(h_ref[...], wneigh_t_ref[...])
    a0 = a0_ref[...]
    out0 = jnp.where(out0 > 0, out0, a0 * out0)
    feat1 = jnp.concatenate([out0, feat_ref[...]], axis=1)
    feat1_ref[...] = feat1
    fb1 = _bn_cols(feat1, g1_ref[...], b1_ref[...])
    q_ref[...] = _dot(fb1, wq_t_ref[...]) + qb_ref[...]
    k_ref[...] = _dot(fb1, wk_t_ref[...])
    v_ref[...] = _dot(fb1, wv_t_ref[...])


def _feat2_body(rst_ref, a1_ref, feat1_ref, feat2_ref):
    rst = rst_ref[...]
    a1 = a1_ref[...]
    out1 = jnp.where(rst > 0, rst, a1 * rst)
    feat2_ref[...] = jnp.concatenate([out1, feat1_ref[...]], axis=1)


def _bn2_body(feat2_ref, g2_ref, b2_ref, wu_t_ref, fb2_ref, fu_ref):
    fb2 = _bn_cols(feat2_ref[...], g2_ref[...], b2_ref[...])
    fb2_ref[...] = fb2
    fu_ref[...] = _dot(fb2, wu_t_ref[...])


def _final_body(srg_ref, wout_t_ref, a2_ref, feat2l_ref, wrf_t_ref,
                rfb_ref, wsr_t_ref, dec_ref, logits_ref):
    srg = _dot(srg_ref[...], wout_t_ref[...])
    a2 = a2_ref[...]
    srg = jnp.where(srg > 0, srg, a2 * srg)
    sr = jnp.concatenate([feat2l_ref[...], srg], axis=1)
    dec_ref[...] = _dot(sr, wrf_t_ref[...]) + rfb_ref[...]
    logits_ref[...] = _dot(sr, wsr_t_ref[...])


def _call(body, out_shapes, *args):
    return pl.pallas_call(
        body,
        out_shape=out_shapes,
    )(*args)


def _seg_softmax(e, seg, num):
    ex = jnp.exp(e - jnp.max(e))
    den = jax.ops.segment_sum(ex, seg, num_segments=num)
    return ex / den[seg]


def kernel(iid, edge_index_mg, edge_index_sg, segment_ids, last_nodes,
           emb_table, bn0_gamma, bn0_beta, gru_Wih, gru_Whh, gru_bih, gru_bhh,
           fc_self_W, fc_neigh_W, prelu0_a, bn1_gamma, bn1_beta,
           fc_q_W, fc_q_b, fc_k_W, fc_v_W, fc_e1_W, prelu1_a,
           bn2_gamma, bn2_beta, fc_u_W, fc_v2_W, fc_v2_b, fc_e2_W, fc_out_W,
           prelu2_a, fc_RF_W, fc_RF_b, fc_sr_W):
    n = iid.shape[0]
    d = emb_table.shape[1]
    b = last_nodes.shape[0]
    f32 = jnp.float32

    def row(x):
        return x.reshape(1, -1).astype(f32)

    feat = emb_table[iid]

    src, dst = edge_index_mg[0], edge_index_mg[1]
    e = src.shape[0]
    order = jnp.argsort(dst)
    src_s = src[order]
    dst_s = dst[order]
    counts = jnp.bincount(dst, length=n)
    offsets = jnp.cumsum(counts) - counts
    rank_s = jnp.arange(e, dtype=jnp.int32) - offsets[dst_s].astype(jnp.int32)
    maxdeg = counts.max()

    fb0, S = _call(
        _pre_body,
        (jax.ShapeDtypeStruct((n, d), f32),
         jax.ShapeDtypeStruct((n, d), f32)),
        feat, row(bn0_gamma), row(bn0_beta), fc_self_W.T.astype(f32))

    blk = _GRU_BLK
    np_pad = ((n + blk - 1) // blk) * blk
    perm = jnp.argsort(-counts)
    nodepos = jnp.zeros((n,), jnp.int32).at[perm].set(
        jnp.arange(n, dtype=jnp.int32))
    counts_p = jnp.zeros((np_pad,), jnp.int32).at[:n].set(counts[perm])
    countsf = counts_p.astype(f32).reshape(np_pad // 128, 128)

    hist = jnp.bincount(counts, length=e + 1)
    k_of_t = n - jnp.cumsum(hist)
    g_tab = jnp.concatenate([jnp.zeros((1,), k_of_t.dtype), jnp.cumsum(k_of_t)])
    pos = g_tab[rank_s].astype(jnp.int32) + nodepos[dst_s]
    src_list = jnp.zeros((e,), jnp.int32).at[pos].set(src_s.astype(jnp.int32))
    x_stream = jnp.zeros((e + blk + 8, d), f32).at[:e].set(fb0[src_list])

    nsteps = maxdeg.astype(jnp.int32).reshape(1)
    h_p = pl.pallas_call(
        _gru_mega_body,
        out_shape=jax.ShapeDtypeStruct((np_pad, d), f32),
        in_specs=[
            pl.BlockSpec(memory_space=pltpu.SMEM),
            pl.BlockSpec(memory_space=pltpu.MemorySpace.VMEM),
            pl.BlockSpec(memory_space=pltpu.MemorySpace.VMEM),
            pl.BlockSpec(memory_space=pltpu.MemorySpace.VMEM),
            pl.BlockSpec(memory_space=pltpu.MemorySpace.VMEM),
            pl.BlockSpec(memory_space=pltpu.MemorySpace.VMEM),
            pl.BlockSpec(memory_space=pltpu.MemorySpace.HBM),
        ],
        out_specs=pl.BlockSpec(memory_space=pltpu.MemorySpace.VMEM),
        scratch_shapes=[
            pltpu.VMEM((2, blk, d), f32),
            pltpu.SemaphoreType.DMA((2,)),
        ],
    )(nsteps, countsf, gru_Wih.T.astype(f32), row(gru_bih),
      gru_Whh.T.astype(f32), row(gru_bhh), x_stream)
    neigh = h_p[nodepos]

    feat1, q, k, v = _call(
        _post_eopa_body,
        (jax.ShapeDtypeStruct((n, 2 * d), f32),
         jax.ShapeDtypeStruct((n, d), f32),
         jax.ShapeDtypeStruct((n, d), f32),
         jax.ShapeDtypeStruct((n, d), f32)),
        S, neigh, fc_neigh_W.T.astype(f32), row(prelu0_a), feat,
        row(bn1_gamma), row(bn1_beta),
        fc_q_W.T.astype(f32), row(fc_q_b),
        fc_k_W.T.astype(f32), fc_v_W.T.astype(f32))

    src2, dst2 = edge_index_sg[0], edge_index_sg[1]
    e = jax.nn.sigmoid(q[src2] + k[dst2]) @ fc_e1_W.T
    a_att = _seg_softmax(e, dst2, n)
    rst = jax.ops.segment_sum(v[src2] * a_att, dst2, num_segments=n)

    feat2 = _call(
        _feat2_body,
        jax.ShapeDtypeStruct((n, 3 * d), f32),
        rst, row(prelu1_a), feat1)
    fb2, fu = _call(
        _bn2_body,
        (jax.ShapeDtypeStruct((n, 3 * d), f32),
         jax.ShapeDtypeStruct((n, d), f32)),
        feat2, row(bn2_gamma), row(bn2_beta), fc_u_W.T.astype(f32))

    fv = fb2[last_nodes] @ fc_v2_W.T + fc_v2_b
    e2 = jax.nn.sigmoid(fu + fv[segment_ids]) @ fc_e2_W.T
    alpha = _seg_softmax(e2, segment_ids, b)
    sr_g = jax.ops.segment_sum(fb2 * alpha, segment_ids, num_segments=b)

    dec, logits = _call(
        _final_body,
        (jax.ShapeDtypeStruct((b, fc_RF_W.shape[0]), f32),
         jax.ShapeDtypeStruct((b, fc_sr_W.shape[0]), f32)),
        sr_g, fc_out_W.T.astype(f32), row(prelu2_a), feat2[last_nodes],
        fc_RF_W.T.astype(f32), row(fc_RF_b), fc_sr_W.T.astype(f32))

    return (dec, logits)


# speedup vs baseline: 6.3331x; 1.7952x over previous
import functools

import jax
import jax.numpy as jnp
from jax import lax
from jax.experimental import pallas as pl
from jax.experimental.pallas import tpu as pltpu
from jax.experimental.pallas import tpu_sc as plsc


def _bn_cols(x, g, b):
    m = jnp.mean(x, axis=0, keepdims=True)
    v = jnp.mean((x - m) ** 2, axis=0, keepdims=True)
    return (x - m) * jax.lax.rsqrt(v + 1e-5) * g + b


def _dot(a, b):
    return jnp.dot(a, b, preferred_element_type=jnp.float32)


def _pre_body(feat_ref, g_ref, b_ref, wself_t_ref, fb_ref, s_ref):
    fb = _bn_cols(feat_ref[...], g_ref[...], b_ref[...])
    fb_ref[...] = fb
    s_ref[...] = _dot(fb, wself_t_ref[...])


_GRU_BLK = 1024


def _gru_mega_body(nsteps_ref, countsf_ref, wih_t_ref, bih_ref, whh_t_ref,
                   bhh_ref, xs_ref, out_ref, xbuf_ref, sem_ref):
    blk = _GRU_BLK
    d = out_ref.shape[1]
    out_ref[...] = jnp.zeros(out_ref.shape, out_ref.dtype)
    nsteps = nsteps_ref[0]
    cf = countsf_ref[...]
    wih = wih_t_ref[...]
    whh = whh_t_ref[...]
    bih = bih_ref[...]
    bhh = bhh_ref[...]

    def dma(slot, start):
        return pltpu.make_async_copy(
            xs_ref.at[pl.ds(start, blk)], xbuf_ref.at[slot], sem_ref.at[slot])

    def step(t, R):
        tf = t.astype(jnp.float32)
        K = jnp.sum((cf > tf).astype(jnp.float32)).astype(jnp.int32)
        nblk = (K + blk - 1) // blk
        dma(0, R).start()

        def body(b, _):
            base = b * blk
            slot = jax.lax.rem(b, 2)

            @pl.when(b + 1 < nblk)
            def _():
                dma(jax.lax.rem(b + 1, 2), R + base + blk).start()

            dma(slot, R + base).wait()
            x = xbuf_ref[slot]
            gi = _dot(x, wih) + bih
            h = out_ref[pl.ds(base, blk), :]
            gh = _dot(h, whh) + bhh
            ir, iz, inew = gi[:, :d], gi[:, d:2 * d], gi[:, 2 * d:]
            hr, hz, hnn = gh[:, :d], gh[:, d:2 * d], gh[:, 2 * d:]
            r = jax.nn.sigmoid(ir + hr)
            z = jax.nn.sigmoid(iz + hz)
            cand = jnp.tanh(inew + r * hnn)
            hn = (1.0 - z) * cand + z * h
            rowid = base + jax.lax.broadcasted_iota(jnp.int32, (blk, 1), 0)
            out_ref[pl.ds(base, blk), :] = jnp.where(rowid < K, hn, h)
            return 0

        jax.lax.fori_loop(0, nblk, body, 0)
        return R + K

    jax.lax.fori_loop(0, nsteps, step, jnp.int32(0))


def _post_eopa_body(s_ref, h_ref, wneigh_t_ref, a0_ref, feat_ref,
                    g1_ref, b1_ref, wq_t_ref, qb_ref, wk_t_ref, wv_t_ref,
                    feat1_ref, q_ref, k_ref, v_ref):
    out0 = s_ref[...] + _dot(h_ref[...], wneigh_t_ref[...])
    a0 = a0_ref[...]
    out0 = jnp.where(out0 > 0, out0, a0 * out0)
    feat1 = jnp.concatenate([out0, feat_ref[...]], axis=1)
    feat1_ref[...] = feat1
    fb1 = _bn_cols(feat1, g1_ref[...], b1_ref[...])
    q_ref[...] = _dot(fb1, wq_t_ref[...]) + qb_ref[...]
    k_ref[...] = _dot(fb1, wk_t_ref[...])
    v_ref[...] = _dot(fb1, wv_t_ref[...])


def _feat2_body(rst_ref, a1_ref, feat1_ref, feat2_ref):
    rst = rst_ref[...]
    a1 = a1_ref[...]
    out1 = jnp.where(rst > 0, rst, a1 * rst)
    feat2_ref[...] = jnp.concatenate([out1, feat1_ref[...]], axis=1)


def _bn2_body(feat2_ref, g2_ref, b2_ref, wu_t_ref, fb2_ref, fu_ref,
              mean_ref, rstd_ref):
    x = feat2_ref[...]
    m = jnp.mean(x, axis=0, keepdims=True)
    v = jnp.mean((x - m) ** 2, axis=0, keepdims=True)
    rs = jax.lax.rsqrt(v + 1e-5)
    mean_ref[...] = m
    rstd_ref[...] = rs
    fb2 = (x - m) * rs * g2_ref[...] + b2_ref[...]
    fb2_ref[...] = fb2
    fu_ref[...] = _dot(fb2, wu_t_ref[...])


def _fv_body(f2l_ref, mean_ref, rstd_ref, g2_ref, b2_ref, wv2_t_ref,
             bv2_ref, fv_ref):
    fb2l = ((f2l_ref[...] - mean_ref[...]) * rstd_ref[...] * g2_ref[...]
            + b2_ref[...])
    fv_ref[...] = _dot(fb2l, wv2_t_ref[...]) + bv2_ref[...]


def _read_a_body(fu_ref, seg_ref, fv_ref, we2_ref, e2_ref):
    nb = fu_ref.shape[0]
    nseg = fv_ref.shape[0]
    oh = (seg_ref[...] ==
          jax.lax.broadcasted_iota(jnp.int32, (nb, nseg), 1)).astype(
              jnp.float32)
    fvn = _dot(oh, fv_ref[...])
    sig = jax.nn.sigmoid(fu_ref[...] + fvn)
    e2_ref[...] = jnp.sum(sig * we2_ref[...], axis=1, keepdims=True)


def _read_b_body(e2_ref, mx_ref, seg_ref, fb2_ref, s_ref, den_ref):
    i = pl.program_id(0)

    @pl.when(i == 0)
    def _():
        s_ref[...] = jnp.zeros(s_ref.shape, s_ref.dtype)
        den_ref[...] = jnp.zeros(den_ref.shape, den_ref.dtype)

    nb = e2_ref.shape[0]
    nseg = s_ref.shape[0]
    ex = jnp.exp(e2_ref[...] - mx_ref[0])
    oh = (seg_ref[...] ==
          jax.lax.broadcasted_iota(jnp.int32, (nb, nseg), 1)).astype(
              jnp.float32)
    w = oh * ex
    dn = (((0,), (0,)), ((), ()))
    s_ref[...] += jax.lax.dot_general(
        w, fb2_ref[...], dn, preferred_element_type=jnp.float32)
    den_ref[...] += jax.lax.dot_general(
        w, jnp.ones((nb, den_ref.shape[1]), jnp.float32), dn,
        preferred_element_type=jnp.float32)


def _final_body(s_ref, den_ref, wout_t_ref, a2_ref, feat2l_ref, wrf_t_ref,
                rfb_ref, wsr_t_ref, dec_ref, logits_ref):
    den0 = den_ref[...][:, :1]
    srg_raw = jnp.where(den0 > 0, s_ref[...] / den0, 0.0)
    srg = _dot(srg_raw, wout_t_ref[...])
    a2 = a2_ref[...]
    srg = jnp.where(srg > 0, srg, a2 * srg)
    sr = jnp.concatenate([feat2l_ref[...], srg], axis=1)
    dec_ref[...] = _dot(sr, wrf_t_ref[...]) + rfb_ref[...]
    logits_ref[...] = _dot(sr, wsr_t_ref[...])


_SC_NW = 32
_SC_CH = 128


def _sc_gather_rows(table, idx):
    m = idx.shape[0]
    d = table.shape[1]
    per_w = m // _SC_NW
    nch = per_w // _SC_CH
    idxm = idx.reshape(m // _SC_CH, _SC_CH)
    mesh = plsc.VectorSubcoreMesh(core_axis_name="c", subcore_axis_name="s")

    @functools.partial(
        pl.kernel, mesh=mesh,
        out_type=jax.ShapeDtypeStruct((m, d), table.dtype),
        scratch_types=[
            pltpu.VMEM((nch, _SC_CH), jnp.int32),
            pltpu.VMEM((2, _SC_CH, d), table.dtype),
            pltpu.SemaphoreType.DMA((2,)),
            pltpu.SemaphoreType.DMA((2,)),
        ],
    )
    def gather_k(table_hbm, idxm_hbm, out_hbm, idx_v, rows_v, gsem, wsem):
        wid = lax.axis_index("s") * 2 + lax.axis_index("c")
        base = wid * per_w
        pltpu.sync_copy(idxm_hbm.at[pl.ds(wid * nch, nch)], idx_v)
        pltpu.async_copy(table_hbm.at[idx_v.at[0]], rows_v.at[0], gsem.at[0])

        def body(c, _):
            slot = lax.rem(c, 2)
            nslot = lax.rem(c + 1, 2)

            @pl.when(c + 1 < nch)
            def _():
                @pl.when(c >= 1)
                def _():
                    pltpu.make_async_copy(
                        rows_v.at[nslot], out_hbm.at[pl.ds(base, _SC_CH)],
                        wsem.at[nslot]).wait()

                pltpu.async_copy(
                    table_hbm.at[idx_v.at[c + 1]], rows_v.at[nslot],
                    gsem.at[nslot])

            pltpu.make_async_copy(
                table_hbm.at[idx_v.at[c]], rows_v.at[slot],
                gsem.at[slot]).wait()
            pltpu.async_copy(
                rows_v.at[slot], out_hbm.at[pl.ds(base + c * _SC_CH, _SC_CH)],
                wsem.at[slot])
            return 0

        lax.fori_loop(0, nch, body, 0, unroll=False)
        for c in range(max(nch - 2, 0), nch):
            pltpu.make_async_copy(
                rows_v.at[c % 2], out_hbm.at[pl.ds(base, _SC_CH)],
                wsem.at[c % 2]).wait()

    return gather_k(table, idxm)


def _sc_build_x(fb0, src_m, rank_m, dst_m, g_tab, nodepos, m_out, dump):
    nch_all, ch = src_m.shape
    m = nch_all * ch
    d = fb0.shape[1]
    per_w = m // _SC_NW
    nch = per_w // ch
    mesh = plsc.VectorSubcoreMesh(core_axis_name="c", subcore_axis_name="s")

    @functools.partial(
        pl.kernel, mesh=mesh,
        out_type=jax.ShapeDtypeStruct((m_out, d), fb0.dtype),
        scratch_types=[
            pltpu.VMEM((nch, ch), jnp.int32),
            pltpu.VMEM((nch, ch), jnp.int32),
            pltpu.VMEM((nch, ch), jnp.int32),
            pltpu.VMEM((ch,), jnp.int32),
            pltpu.VMEM((ch,), jnp.int32),
            pltpu.VMEM((2, ch), jnp.int32),
            pltpu.VMEM((2, ch, d), jnp.float32),
            pltpu.SemaphoreType.DMA((2,)),
            pltpu.SemaphoreType.DMA,
            pltpu.SemaphoreType.DMA((2,)),
        ],
    )
    def build_k(fb_hbm, srcm_hbm, rankm_hbm, dstm_hbm, gt_hbm, np_hbm,
                out_hbm, src_v, rank_v, dst_v, gt_v, np_v, pos_v, rows_v,
                gsem, ssem, wsem):
        wid = lax.axis_index("s") * 2 + lax.axis_index("c")
        pltpu.sync_copy(srcm_hbm.at[pl.ds(wid * nch, nch)], src_v)
        pltpu.sync_copy(rankm_hbm.at[pl.ds(wid * nch, nch)], rank_v)
        pltpu.sync_copy(dstm_hbm.at[pl.ds(wid * nch, nch)], dst_v)
        pltpu.async_copy(fb_hbm.at[src_v.at[0]], rows_v.at[0], gsem.at[0])

        def body(c, _):
            slot = lax.rem(c, 2)
            nslot = lax.rem(c + 1, 2)

            @pl.when(c + 1 < nch)
            def _():
                @pl.when(c >= 1)
                def _():
                    pltpu.make_async_copy(rows_v.at[nslot],
                                          out_hbm.at[pos_v.at[nslot]],
                                          wsem.at[nslot]).wait()

                pltpu.async_copy(fb_hbm.at[src_v.at[c + 1]],
                                 rows_v.at[nslot], gsem.at[nslot])

            cp_g = pltpu.async_copy(gt_hbm.at[rank_v.at[c]], gt_v, ssem)
            cp_n = pltpu.async_copy(np_hbm.at[dst_v.at[c]], np_v, ssem)
            cp_g.wait()
            cp_n.wait()
            for s in (0, 1):
                @pl.when(slot == s)
                def _():
                    for i in range(ch // 16):
                        sl = pl.ds(i * 16, 16)
                        pos_v[s, sl] = jnp.minimum(gt_v[sl] + np_v[sl], dump)

            pltpu.make_async_copy(fb_hbm.at[src_v.at[c]], rows_v.at[slot],
                                  gsem.at[slot]).wait()
            pltpu.async_copy(rows_v.at[slot], out_hbm.at[pos_v.at[slot]],
                             wsem.at[slot])
            return 0

        lax.fori_loop(0, nch, body, 0, unroll=False)
        for c in range(max(nch - 2, 0), nch):
            pltpu.make_async_copy(rows_v.at[c % 2],
                                  out_hbm.at[pos_v.at[c % 2]],
                                  wsem.at[c % 2]).wait()

    return build_k(fb0, src_m, rank_m, dst_m, g_tab, nodepos)


def _edge_e_body(qs_ref, kd_ref, w_ref, out_ref):
    sig = jax.nn.sigmoid(qs_ref[...] + kd_ref[...])
    out_ref[...] = jnp.sum(sig * w_ref[...], axis=1, keepdims=True)


def _call(body, out_shapes, *args):
    return pl.pallas_call(
        body,
        out_shape=out_shapes,
    )(*args)


def _seg_softmax(e, seg, num):
    ex = jnp.exp(e - jnp.max(e))
    den = jax.ops.segment_sum(ex, seg, num_segments=num)
    return ex / den[seg]


def kernel(iid, edge_index_mg, edge_index_sg, segment_ids, last_nodes,
           emb_table, bn0_gamma, bn0_beta, gru_Wih, gru_Whh, gru_bih, gru_bhh,
           fc_self_W, fc_neigh_W, prelu0_a, bn1_gamma, bn1_beta,
           fc_q_W, fc_q_b, fc_k_W, fc_v_W, fc_e1_W, prelu1_a,
           bn2_gamma, bn2_beta, fc_u_W, fc_v2_W, fc_v2_b, fc_e2_W, fc_out_W,
           prelu2_a, fc_RF_W, fc_RF_b, fc_sr_W):
    n = iid.shape[0]
    d = emb_table.shape[1]
    b = last_nodes.shape[0]
    f32 = jnp.float32

    def row(x):
        return x.reshape(1, -1).astype(f32)

    feat = emb_table[iid]

    src, dst = edge_index_mg[0], edge_index_mg[1]
    e = src.shape[0]
    order = jnp.argsort(dst)
    src_s = src[order].astype(jnp.int32)
    dst_s = dst[order].astype(jnp.int32)
    counts = jnp.bincount(dst, length=n)
    ar = jnp.arange(e, dtype=jnp.int32)
    is_start = jnp.concatenate(
        [jnp.ones((1,), jnp.bool_), dst_s[1:] != dst_s[:-1]])
    seg_start = jax.lax.cummax(jnp.where(is_start, ar, 0))
    rank_s = ar - seg_start
    maxdeg = counts.max()

    fb0, S = _call(
        _pre_body,
        (jax.ShapeDtypeStruct((n, d), f32),
         jax.ShapeDtypeStruct((n, d), f32)),
        feat, row(bn0_gamma), row(bn0_beta), fc_self_W.T.astype(f32))

    blk = _GRU_BLK
    np_pad = ((n + blk - 1) // blk) * blk
    perm = jnp.argsort(-counts)
    nodepos = jnp.zeros((n,), jnp.int32).at[perm].set(
        jnp.arange(n, dtype=jnp.int32))
    counts_p = jnp.zeros((np_pad,), jnp.int32).at[:n].set(counts[perm])
    countsf = counts_p.astype(f32).reshape(np_pad // 128, 128)

    hist = jnp.bincount(counts, length=e + 1)
    k_of_t = n - jnp.cumsum(hist)
    g_tab = jnp.concatenate(
        [jnp.zeros((1,), jnp.int32),
         jnp.cumsum(k_of_t).astype(jnp.int32)])
    m_e = ((e + 4095) // 4096) * 4096
    m_g = ((e + blk + 8 + 4095) // 4096) * 4096 + 4096

    def padm(x, fill):
        return jnp.full((m_e,), fill, jnp.int32).at[:e].set(x).reshape(
            m_e // _SC_CH, _SC_CH)

    x_stream = _sc_build_x(
        fb0, padm(src_s, 0), padm(rank_s, e + 1), padm(dst_s, 0),
        g_tab, nodepos, m_g, m_g - 1)

    nsteps = maxdeg.astype(jnp.int32).reshape(1)
    h_p = pl.pallas_call(
        _gru_mega_body,
        out_shape=jax.ShapeDtypeStruct((np_pad, d), f32),
        in_specs=[
            pl.BlockSpec(memory_space=pltpu.SMEM),
            pl.BlockSpec(memory_space=pltpu.MemorySpace.VMEM),
            pl.BlockSpec(memory_space=pltpu.MemorySpace.VMEM),
            pl.BlockSpec(memory_space=pltpu.MemorySpace.VMEM),
            pl.BlockSpec(memory_space=pltpu.MemorySpace.VMEM),
            pl.BlockSpec(memory_space=pltpu.MemorySpace.VMEM),
            pl.BlockSpec(memory_space=pltpu.MemorySpace.HBM),
        ],
        out_specs=pl.BlockSpec(memory_space=pltpu.MemorySpace.VMEM),
        scratch_shapes=[
            pltpu.VMEM((2, blk, d), f32),
            pltpu.SemaphoreType.DMA((2,)),
        ],
    )(nsteps, countsf, gru_Wih.T.astype(f32), row(gru_bih),
      gru_Whh.T.astype(f32), row(gru_bhh), x_stream)
    m_n = ((n + 32767) // 32768) * 32768
    nodepos_p = jnp.zeros((m_n,), jnp.int32).at[:n].set(nodepos)
    neigh = _sc_gather_rows(h_p, nodepos_p)[:n]

    feat1, q, k, v = _call(
        _post_eopa_body,
        (jax.ShapeDtypeStruct((n, 2 * d), f32),
         jax.ShapeDtypeStruct((n, d), f32),
         jax.ShapeDtypeStruct((n, d), f32),
         jax.ShapeDtypeStruct((n, d), f32)),
        S, neigh, fc_neigh_W.T.astype(f32), row(prelu0_a), feat,
        row(bn1_gamma), row(bn1_beta),
        fc_q_W.T.astype(f32), row(fc_q_b),
        fc_k_W.T.astype(f32), fc_v_W.T.astype(f32))

    src2 = edge_index_sg[0].astype(jnp.int32)
    dst2 = edge_index_sg[1].astype(jnp.int32)
    src2_p = jnp.zeros((m_e,), jnp.int32).at[:e].set(src2)
    dst2_p = jnp.zeros((m_e,), jnp.int32).at[:e].set(dst2)
    qs = _sc_gather_rows(q, src2_p)
    kd = _sc_gather_rows(k, dst2_p)
    vs = _sc_gather_rows(v, src2_p)

    eblk = 8192
    e_att = pl.pallas_call(
        _edge_e_body,
        grid=(m_e // eblk,),
        in_specs=[
            pl.BlockSpec((eblk, d), lambda i: (i, 0)),
            pl.BlockSpec((eblk, d), lambda i: (i, 0)),
            pl.BlockSpec((1, d), lambda i: (0, 0)),
        ],
        out_specs=pl.BlockSpec((eblk, 1), lambda i: (i, 0)),
        out_shape=jax.ShapeDtypeStruct((m_e, 1), f32),
    )(qs, kd, fc_e1_W.astype(f32))
    e_att = e_att[:e, 0]
    ex = jnp.exp(e_att - jnp.max(e_att))
    den = jax.ops.segment_sum(ex, dst2, num_segments=n)
    rst_u = jax.ops.segment_sum(vs[:e] * ex[:, None], dst2, num_segments=n)
    rst = jnp.where(den[:, None] > 0, rst_u / den[:, None], 0.0)

    feat2 = _call(
        _feat2_body,
        jax.ShapeDtypeStruct((n, 3 * d), f32),
        rst, row(prelu1_a), feat1)
    fb2, fu, mean2, rstd2 = _call(
        _bn2_body,
        (jax.ShapeDtypeStruct((n, 3 * d), f32),
         jax.ShapeDtypeStruct((n, d), f32),
         jax.ShapeDtypeStruct((1, 3 * d), f32),
         jax.ShapeDtypeStruct((1, 3 * d), f32)),
        feat2, row(bn2_gamma), row(bn2_beta), fc_u_W.T.astype(f32))

    feat2_last = feat2[last_nodes]
    fv = _call(
        _fv_body, jax.ShapeDtypeStruct((b, d), f32),
        feat2_last, mean2, rstd2, row(bn2_gamma), row(bn2_beta),
        fc_v2_W.T.astype(f32), row(fc_v2_b))

    segc = segment_ids.astype(jnp.int32).reshape(n, 1)
    rblk = 2000
    e2 = pl.pallas_call(
        _read_a_body,
        grid=(n // rblk,),
        in_specs=[
            pl.BlockSpec((rblk, d), lambda i: (i, 0)),
            pl.BlockSpec((rblk, 1), lambda i: (i, 0)),
            pl.BlockSpec((b, d), lambda i: (0, 0)),
            pl.BlockSpec((1, d), lambda i: (0, 0)),
        ],
        out_specs=pl.BlockSpec((rblk, 1), lambda i: (i, 0)),
        out_shape=jax.ShapeDtypeStruct((n, 1), f32),
    )(fu, segc, fv, fc_e2_W.astype(f32))

    mx = jnp.max(e2).reshape(1)
    s_acc, den = pl.pallas_call(
        _read_b_body,
        grid=(n // rblk,),
        in_specs=[
            pl.BlockSpec((rblk, 1), lambda i: (i, 0)),
            pl.BlockSpec(memory_space=pltpu.SMEM),
            pl.BlockSpec((rblk, 1), lambda i: (i, 0)),
            pl.BlockSpec((rblk, 3 * d), lambda i: (i, 0)),
        ],
        out_specs=(pl.BlockSpec((b, 3 * d), lambda i: (0, 0)),
                   pl.BlockSpec((b, 8), lambda i: (0, 0))),
        out_shape=(jax.ShapeDtypeStruct((b, 3 * d), f32),
                   jax.ShapeDtypeStruct((b, 8), f32)),
    )(e2, mx, segc, fb2)

    dec, logits = _call(
        _final_body,
        (jax.ShapeDtypeStruct((b, fc_RF_W.shape[0]), f32),
         jax.ShapeDtypeStruct((b, fc_sr_W.shape[0]), f32)),
        s_acc, den, fc_out_W.T.astype(f32), row(prelu2_a), feat2_last,
        fc_RF_W.T.astype(f32), row(fc_RF_b), fc_sr_W.T.astype(f32))

    return (dec, logits)


# speedup vs baseline: 6.3523x; 1.0030x over previous
import functools

import jax
import jax.numpy as jnp
from jax import lax
from jax.experimental import pallas as pl
from jax.experimental.pallas import tpu as pltpu
from jax.experimental.pallas import tpu_sc as plsc


def _bn_cols(x, g, b):
    m = jnp.mean(x, axis=0, keepdims=True)
    v = jnp.mean((x - m) ** 2, axis=0, keepdims=True)
    return (x - m) * jax.lax.rsqrt(v + 1e-5) * g + b


def _dot(a, b):
    return jnp.dot(a, b, preferred_element_type=jnp.float32)


def _pre_body(feat_ref, g_ref, b_ref, wself_t_ref, fb_ref, s_ref):
    fb = _bn_cols(feat_ref[...], g_ref[...], b_ref[...])
    fb_ref[...] = fb
    s_ref[...] = _dot(fb, wself_t_ref[...])


_GRU_BLK = 1024


def _gru_mega_body(nsteps_ref, countsf_ref, wih_t_ref, bih_ref, whh_t_ref,
                   bhh_ref, xs_ref, out_ref, xbuf_ref, sem_ref):
    blk = _GRU_BLK
    d = out_ref.shape[1]
    out_ref[...] = jnp.zeros(out_ref.shape, out_ref.dtype)
    nsteps = nsteps_ref[0]
    cf = countsf_ref[...]
    wih = wih_t_ref[...]
    whh = whh_t_ref[...]
    bih = bih_ref[...]
    bhh = bhh_ref[...]

    def dma(slot, start):
        return pltpu.make_async_copy(
            xs_ref.at[pl.ds(start, blk)], xbuf_ref.at[slot], sem_ref.at[slot])

    def step(t, R):
        tf = t.astype(jnp.float32)
        K = jnp.sum((cf > tf).astype(jnp.float32)).astype(jnp.int32)
        nblk = (K + blk - 1) // blk
        dma(0, R).start()

        def body(b, _):
            base = b * blk
            slot = jax.lax.rem(b, 2)

            @pl.when(b + 1 < nblk)
            def _():
                dma(jax.lax.rem(b + 1, 2), R + base + blk).start()

            dma(slot, R + base).wait()
            x = xbuf_ref[slot]
            gi = _dot(x, wih) + bih
            h = out_ref[pl.ds(base, blk), :]
            gh = _dot(h, whh) + bhh
            ir, iz, inew = gi[:, :d], gi[:, d:2 * d], gi[:, 2 * d:]
            hr, hz, hnn = gh[:, :d], gh[:, d:2 * d], gh[:, 2 * d:]
            r = jax.nn.sigmoid(ir + hr)
            z = jax.nn.sigmoid(iz + hz)
            cand = jnp.tanh(inew + r * hnn)
            hn = (1.0 - z) * cand + z * h
            rowid = base + jax.lax.broadcasted_iota(jnp.int32, (blk, 1), 0)
            out_ref[pl.ds(base, blk), :] = jnp.where(rowid < K, hn, h)
            return 0

        jax.lax.fori_loop(0, nblk, body, 0)
        return R + K

    jax.lax.fori_loop(0, nsteps, step, jnp.int32(0))


def _post_eopa_body(s_ref, h_ref, wneigh_t_ref, a0_ref, feat_ref,
                    g1_ref, b1_ref, wq_t_ref, qb_ref, wk_t_ref, wv_t_ref,
                    feat1_ref, q_ref, k_ref, v_ref):
    out0 = s_ref[...] + _dot(h_ref[...], wneigh_t_ref[...])
    a0 = a0_ref[...]
    out0 = jnp.where(out0 > 0, out0, a0 * out0)
    feat1 = jnp.concatenate([out0, feat_ref[...]], axis=1)
    feat1_ref[...] = feat1
    fb1 = _bn_cols(feat1, g1_ref[...], b1_ref[...])
    q_ref[...] = _dot(fb1, wq_t_ref[...]) + qb_ref[...]
    k_ref[...] = _dot(fb1, wk_t_ref[...])
    v_ref[...] = _dot(fb1, wv_t_ref[...])


def _feat2_body(rst_ref, a1_ref, feat1_ref, feat2_ref):
    rst = rst_ref[...]
    a1 = a1_ref[...]
    out1 = jnp.where(rst > 0, rst, a1 * rst)
    feat2_ref[...] = jnp.concatenate([out1, feat1_ref[...]], axis=1)


def _bn2_body(feat2_ref, g2_ref, b2_ref, wu_t_ref, fb2_ref, fu_ref,
              mean_ref, rstd_ref):
    x = feat2_ref[...]
    m = jnp.mean(x, axis=0, keepdims=True)
    v = jnp.mean((x - m) ** 2, axis=0, keepdims=True)
    rs = jax.lax.rsqrt(v + 1e-5)
    mean_ref[...] = m
    rstd_ref[...] = rs
    fb2 = (x - m) * rs * g2_ref[...] + b2_ref[...]
    fb2_ref[...] = fb2
    fu_ref[...] = _dot(fb2, wu_t_ref[...])


def _fv_body(f2l_ref, mean_ref, rstd_ref, g2_ref, b2_ref, wv2_t_ref,
             bv2_ref, fv_ref):
    fb2l = ((f2l_ref[...] - mean_ref[...]) * rstd_ref[...] * g2_ref[...]
            + b2_ref[...])
    fv_ref[...] = _dot(fb2l, wv2_t_ref[...]) + bv2_ref[...]


def _read_a_body(fu_ref, seg_ref, fv_ref, we2_ref, e2_ref):
    nb = fu_ref.shape[0]
    nseg = fv_ref.shape[0]
    oh = (seg_ref[...] ==
          jax.lax.broadcasted_iota(jnp.int32, (nb, nseg), 1)).astype(
              jnp.float32)
    fvn = _dot(oh, fv_ref[...])
    sig = jax.nn.sigmoid(fu_ref[...] + fvn)
    e2_ref[...] = jnp.sum(sig * we2_ref[...], axis=1, keepdims=True)


def _read_b_body(e2_ref, mx_ref, seg_ref, fb2_ref, s_ref, den_ref):
    i = pl.program_id(0)

    @pl.when(i == 0)
    def _():
        s_ref[...] = jnp.zeros(s_ref.shape, s_ref.dtype)
        den_ref[...] = jnp.zeros(den_ref.shape, den_ref.dtype)

    nb = e2_ref.shape[0]
    nseg = s_ref.shape[0]
    ex = jnp.exp(e2_ref[...] - mx_ref[0])
    oh = (seg_ref[...] ==
          jax.lax.broadcasted_iota(jnp.int32, (nb, nseg), 1)).astype(
              jnp.float32)
    w = oh * ex
    dn = (((0,), (0,)), ((), ()))
    s_ref[...] += jax.lax.dot_general(
        w, fb2_ref[...], dn, preferred_element_type=jnp.float32)
    den_ref[...] += jax.lax.dot_general(
        w, jnp.ones((nb, den_ref.shape[1]), jnp.float32), dn,
        preferred_element_type=jnp.float32)


def _final_body(s_ref, den_ref, wout_t_ref, a2_ref, feat2l_ref, wrf_t_ref,
                rfb_ref, wsr_t_ref, dec_ref, logits_ref):
    den0 = den_ref[...][:, :1]
    srg_raw = jnp.where(den0 > 0, s_ref[...] / den0, 0.0)
    srg = _dot(srg_raw, wout_t_ref[...])
    a2 = a2_ref[...]
    srg = jnp.where(srg > 0, srg, a2 * srg)
    sr = jnp.concatenate([feat2l_ref[...], srg], axis=1)
    dec_ref[...] = _dot(sr, wrf_t_ref[...]) + rfb_ref[...]
    logits_ref[...] = _dot(sr, wsr_t_ref[...])


_SC_NW = 32
_SC_CH = 128


def _sc_gather_rows(table, idx):
    m = idx.shape[0]
    d = table.shape[1]
    per_w = m // _SC_NW
    nch = per_w // _SC_CH
    idxm = idx.reshape(m // _SC_CH, _SC_CH)
    mesh = plsc.VectorSubcoreMesh(core_axis_name="c", subcore_axis_name="s")

    nbuf = 4

    @functools.partial(
        pl.kernel, mesh=mesh,
        out_type=jax.ShapeDtypeStruct((m, d), table.dtype),
        scratch_types=[
            pltpu.VMEM((nch, _SC_CH), jnp.int32),
            pltpu.VMEM((nbuf, _SC_CH, d), table.dtype),
            pltpu.SemaphoreType.DMA((nbuf,)),
            pltpu.SemaphoreType.DMA((nbuf,)),
        ],
    )
    def gather_k(table_hbm, idxm_hbm, out_hbm, idx_v, rows_v, gsem, wsem):
        wid = lax.axis_index("s") * 2 + lax.axis_index("c")
        base = wid * per_w
        pltpu.sync_copy(idxm_hbm.at[pl.ds(wid * nch, nch)], idx_v)
        for j in range(nbuf - 1):
            if j < nch:
                pltpu.async_copy(table_hbm.at[idx_v.at[j]], rows_v.at[j],
                                 gsem.at[j])

        def body(c, _):
            slot = lax.rem(c, nbuf)
            pltpu.make_async_copy(
                table_hbm.at[idx_v.at[c]], rows_v.at[slot],
                gsem.at[slot]).wait()
            pltpu.async_copy(
                rows_v.at[slot], out_hbm.at[pl.ds(base + c * _SC_CH, _SC_CH)],
                wsem.at[slot])

            @pl.when(c + nbuf - 1 < nch)
            def _():
                ns = lax.rem(c + nbuf - 1, nbuf)

                @pl.when(c >= 1)
                def _():
                    pltpu.make_async_copy(
                        rows_v.at[ns], out_hbm.at[pl.ds(base, _SC_CH)],
                        wsem.at[ns]).wait()

                pltpu.async_copy(
                    table_hbm.at[idx_v.at[c + nbuf - 1]], rows_v.at[ns],
                    gsem.at[ns])

            return 0

        lax.fori_loop(0, nch, body, 0, unroll=False)
        for c in range(max(nch - nbuf, 0), nch):
            pltpu.make_async_copy(
                rows_v.at[c % nbuf], out_hbm.at[pl.ds(base, _SC_CH)],
                wsem.at[c % nbuf]).wait()

    return gather_k(table, idxm)


def _sc_build_x(fb0, src_m, rank_m, dst_m, g_tab, nodepos, m_out, dump):
    nch_all, ch = src_m.shape
    m = nch_all * ch
    d = fb0.shape[1]
    per_w = m // _SC_NW
    nch = per_w // ch
    mesh = plsc.VectorSubcoreMesh(core_axis_name="c", subcore_axis_name="s")

    @functools.partial(
        pl.kernel, mesh=mesh,
        out_type=jax.ShapeDtypeStruct((m_out, d), fb0.dtype),
        scratch_types=[
            pltpu.VMEM((nch, ch), jnp.int32),
            pltpu.VMEM((nch, ch), jnp.int32),
            pltpu.VMEM((nch, ch), jnp.int32),
            pltpu.VMEM((2, ch), jnp.int32),
            pltpu.VMEM((2, ch), jnp.int32),
            pltpu.VMEM((2, ch), jnp.int32),
            pltpu.VMEM((2, ch, d), jnp.float32),
            pltpu.SemaphoreType.DMA((2,)),
            pltpu.SemaphoreType.DMA((2,)),
            pltpu.SemaphoreType.DMA((2,)),
            pltpu.SemaphoreType.DMA((2,)),
        ],
    )
    def build_k(fb_hbm, srcm_hbm, rankm_hbm, dstm_hbm, gt_hbm, np_hbm,
                out_hbm, src_v, rank_v, dst_v, gt_v, np_v, pos_v, rows_v,
                gsem, s1sem, s2sem, wsem):
        wid = lax.axis_index("s") * 2 + lax.axis_index("c")
        pltpu.sync_copy(srcm_hbm.at[pl.ds(wid * nch, nch)], src_v)
        pltpu.sync_copy(rankm_hbm.at[pl.ds(wid * nch, nch)], rank_v)
        pltpu.sync_copy(dstm_hbm.at[pl.ds(wid * nch, nch)], dst_v)
        pltpu.async_copy(fb_hbm.at[src_v.at[0]], rows_v.at[0], gsem.at[0])
        pltpu.async_copy(gt_hbm.at[rank_v.at[0]], gt_v.at[0], s1sem.at[0])
        pltpu.async_copy(np_hbm.at[dst_v.at[0]], np_v.at[0], s2sem.at[0])

        def body(c, _):
            slot = lax.rem(c, 2)
            nslot = lax.rem(c + 1, 2)

            @pl.when(c + 1 < nch)
            def _():
                @pl.when(c >= 1)
                def _():
                    pltpu.make_async_copy(rows_v.at[nslot],
                                          out_hbm.at[pos_v.at[nslot]],
                                          wsem.at[nslot]).wait()

                pltpu.async_copy(fb_hbm.at[src_v.at[c + 1]],
                                 rows_v.at[nslot], gsem.at[nslot])
                pltpu.async_copy(gt_hbm.at[rank_v.at[c + 1]],
                                 gt_v.at[nslot], s1sem.at[nslot])
                pltpu.async_copy(np_hbm.at[dst_v.at[c + 1]],
                                 np_v.at[nslot], s2sem.at[nslot])

            pltpu.make_async_copy(gt_hbm.at[rank_v.at[c]], gt_v.at[slot],
                                  s1sem.at[slot]).wait()
            pltpu.make_async_copy(np_hbm.at[dst_v.at[c]], np_v.at[slot],
                                  s2sem.at[slot]).wait()
            for s in (0, 1):
                @pl.when(slot == s)
                def _():
                    for i in range(ch // 16):
                        sl = pl.ds(i * 16, 16)
                        pos_v[s, sl] = jnp.minimum(
                            gt_v[s, sl] + np_v[s, sl], dump)

            pltpu.make_async_copy(fb_hbm.at[src_v.at[c]], rows_v.at[slot],
                                  gsem.at[slot]).wait()
            pltpu.async_copy(rows_v.at[slot], out_hbm.at[pos_v.at[slot]],
                             wsem.at[slot])
            return 0

        lax.fori_loop(0, nch, body, 0, unroll=False)
        for c in range(max(nch - 2, 0), nch):
            pltpu.make_async_copy(rows_v.at[c % 2],
                                  out_hbm.at[pos_v.at[c % 2]],
                                  wsem.at[c % 2]).wait()

    return build_k(fb0, src_m, rank_m, dst_m, g_tab, nodepos)


def _edge_e_body(qs_ref, kd_ref, w_ref, out_ref):
    sig = jax.nn.sigmoid(qs_ref[...] + kd_ref[...])
    out_ref[...] = jnp.sum(sig * w_ref[...], axis=1, keepdims=True)


def _call(body, out_shapes, *args):
    return pl.pallas_call(
        body,
        out_shape=out_shapes,
    )(*args)


def _seg_softmax(e, seg, num):
    ex = jnp.exp(e - jnp.max(e))
    den = jax.ops.segment_sum(ex, seg, num_segments=num)
    return ex / den[seg]


def kernel(iid, edge_index_mg, edge_index_sg, segment_ids, last_nodes,
           emb_table, bn0_gamma, bn0_beta, gru_Wih, gru_Whh, gru_bih, gru_bhh,
           fc_self_W, fc_neigh_W, prelu0_a, bn1_gamma, bn1_beta,
           fc_q_W, fc_q_b, fc_k_W, fc_v_W, fc_e1_W, prelu1_a,
           bn2_gamma, bn2_beta, fc_u_W, fc_v2_W, fc_v2_b, fc_e2_W, fc_out_W,
           prelu2_a, fc_RF_W, fc_RF_b, fc_sr_W):
    n = iid.shape[0]
    d = emb_table.shape[1]
    b = last_nodes.shape[0]
    f32 = jnp.float32

    def row(x):
        return x.reshape(1, -1).astype(f32)

    feat = emb_table[iid]

    src, dst = edge_index_mg[0], edge_index_mg[1]
    e = src.shape[0]
    order = jnp.argsort(dst)
    src_s = src[order].astype(jnp.int32)
    dst_s = dst[order].astype(jnp.int32)
    counts = jnp.bincount(dst, length=n)
    ar = jnp.arange(e, dtype=jnp.int32)
    is_start = jnp.concatenate(
        [jnp.ones((1,), jnp.bool_), dst_s[1:] != dst_s[:-1]])
    seg_start = jax.lax.cummax(jnp.where(is_start, ar, 0))
    rank_s = ar - seg_start
    maxdeg = counts.max()

    fb0, S = _call(
        _pre_body,
        (jax.ShapeDtypeStruct((n, d), f32),
         jax.ShapeDtypeStruct((n, d), f32)),
        feat, row(bn0_gamma), row(bn0_beta), fc_self_W.T.astype(f32))

    blk = _GRU_BLK
    np_pad = ((n + blk - 1) // blk) * blk
    perm = jnp.argsort(-counts, stable=False)
    nodepos = jnp.zeros((n,), jnp.int32).at[perm].set(
        jnp.arange(n, dtype=jnp.int32))
    counts_p = jnp.zeros((np_pad,), jnp.int32).at[:n].set(counts[perm])
    countsf = counts_p.astype(f32).reshape(np_pad // 128, 128)

    hist = jnp.bincount(counts, length=e + 1)
    k_of_t = n - jnp.cumsum(hist)
    g_tab = jnp.concatenate(
        [jnp.zeros((1,), jnp.int32),
         jnp.cumsum(k_of_t).astype(jnp.int32)])
    m_e = ((e + 4095) // 4096) * 4096
    m_g = ((e + blk + 8 + 4095) // 4096) * 4096 + 4096

    def padm(x, fill):
        return jnp.full((m_e,), fill, jnp.int32).at[:e].set(x).reshape(
            m_e // _SC_CH, _SC_CH)

    x_stream = _sc_build_x(
        fb0, padm(src_s, 0), padm(rank_s, e + 1), padm(dst_s, 0),
        g_tab, nodepos, m_g, m_g - 1)

    nsteps = maxdeg.astype(jnp.int32).reshape(1)
    h_p = pl.pallas_call(
        _gru_mega_body,
        out_shape=jax.ShapeDtypeStruct((np_pad, d), f32),
        in_specs=[
            pl.BlockSpec(memory_space=pltpu.SMEM),
            pl.BlockSpec(memory_space=pltpu.MemorySpace.VMEM),
            pl.BlockSpec(memory_space=pltpu.MemorySpace.VMEM),
            pl.BlockSpec(memory_space=pltpu.MemorySpace.VMEM),
            pl.BlockSpec(memory_space=pltpu.MemorySpace.VMEM),
            pl.BlockSpec(memory_space=pltpu.MemorySpace.VMEM),
            pl.BlockSpec(memory_space=pltpu.MemorySpace.HBM),
        ],
        out_specs=pl.BlockSpec(memory_space=pltpu.MemorySpace.VMEM),
        scratch_shapes=[
            pltpu.VMEM((2, blk, d), f32),
            pltpu.SemaphoreType.DMA((2,)),
        ],
    )(nsteps, countsf, gru_Wih.T.astype(f32), row(gru_bih),
      gru_Whh.T.astype(f32), row(gru_bhh), x_stream)
    m_n = ((n + 32767) // 32768) * 32768
    nodepos_p = jnp.zeros((m_n,), jnp.int32).at[:n].set(nodepos)
    neigh = _sc_gather_rows(h_p, nodepos_p)[:n]

    feat1, q, k, v = _call(
        _post_eopa_body,
        (jax.ShapeDtypeStruct((n, 2 * d), f32),
         jax.ShapeDtypeStruct((n, d), f32),
         jax.ShapeDtypeStruct((n, d), f32),
         jax.ShapeDtypeStruct((n, d), f32)),
        S, neigh, fc_neigh_W.T.astype(f32), row(prelu0_a), feat,
        row(bn1_gamma), row(bn1_beta),
        fc_q_W.T.astype(f32), row(fc_q_b),
        fc_k_W.T.astype(f32), fc_v_W.T.astype(f32))

    src2 = edge_index_sg[0].astype(jnp.int32)
    dst2 = edge_index_sg[1].astype(jnp.int32)
    src2_p = jnp.zeros((m_e,), jnp.int32).at[:e].set(src2)
    dst2_p = jnp.zeros((m_e,), jnp.int32).at[:e].set(dst2)
    qs = _sc_gather_rows(q, src2_p)
    kd = _sc_gather_rows(k, dst2_p)
    vs = _sc_gather_rows(v, src2_p)

    eblk = 8192
    e_att = pl.pallas_call(
        _edge_e_body,
        grid=(m_e // eblk,),
        in_specs=[
            pl.BlockSpec((eblk, d), lambda i: (i, 0)),
            pl.BlockSpec((eblk, d), lambda i: (i, 0)),
            pl.BlockSpec((1, d), lambda i: (0, 0)),
        ],
        out_specs=pl.BlockSpec((eblk, 1), lambda i: (i, 0)),
        out_shape=jax.ShapeDtypeStruct((m_e, 1), f32),
    )(qs, kd, fc_e1_W.astype(f32))
    e_att = e_att[:e, 0]
    ex = jnp.exp(e_att - jnp.max(e_att))
    den = jax.ops.segment_sum(ex, dst2, num_segments=n)
    rst_u = jax.ops.segment_sum(vs[:e] * ex[:, None], dst2, num_segments=n)
    rst = jnp.where(den[:, None] > 0, rst_u / den[:, None], 0.0)

    feat2 = _call(
        _feat2_body,
        jax.ShapeDtypeStruct((n, 3 * d), f32),
        rst, row(prelu1_a), feat1)
    fb2, fu, mean2, rstd2 = _call(
        _bn2_body,
        (jax.ShapeDtypeStruct((n, 3 * d), f32),
         jax.ShapeDtypeStruct((n, d), f32),
         jax.ShapeDtypeStruct((1, 3 * d), f32),
         jax.ShapeDtypeStruct((1, 3 * d), f32)),
        feat2, row(bn2_gamma), row(bn2_beta), fc_u_W.T.astype(f32))

    feat2_last = feat2[last_nodes]
    fv = _call(
        _fv_body, jax.ShapeDtypeStruct((b, d), f32),
        feat2_last, mean2, rstd2, row(bn2_gamma), row(bn2_beta),
        fc_v2_W.T.astype(f32), row(fc_v2_b))

    segc = segment_ids.astype(jnp.int32).reshape(n, 1)
    rblk = 2000
    e2 = pl.pallas_call(
        _read_a_body,
        grid=(n // rblk,),
        in_specs=[
            pl.BlockSpec((rblk, d), lambda i: (i, 0)),
            pl.BlockSpec((rblk, 1), lambda i: (i, 0)),
            pl.BlockSpec((b, d), lambda i: (0, 0)),
            pl.BlockSpec((1, d), lambda i: (0, 0)),
        ],
        out_specs=pl.BlockSpec((rblk, 1), lambda i: (i, 0)),
        out_shape=jax.ShapeDtypeStruct((n, 1), f32),
    )(fu, segc, fv, fc_e2_W.astype(f32))

    mx = jnp.max(e2).reshape(1)
    s_acc, den = pl.pallas_call(
        _read_b_body,
        grid=(n // rblk,),
        in_specs=[
            pl.BlockSpec((rblk, 1), lambda i: (i, 0)),
            pl.BlockSpec(memory_space=pltpu.SMEM),
            pl.BlockSpec((rblk, 1), lambda i: (i, 0)),
            pl.BlockSpec((rblk, 3 * d), lambda i: (i, 0)),
        ],
        out_specs=(pl.BlockSpec((b, 3 * d), lambda i: (0, 0)),
                   pl.BlockSpec((b, 8), lambda i: (0, 0))),
        out_shape=(jax.ShapeDtypeStruct((b, 3 * d), f32),
                   jax.ShapeDtypeStruct((b, 8), f32)),
    )(e2, mx, segc, fb2)

    dec, logits = _call(
        _final_body,
        (jax.ShapeDtypeStruct((b, fc_RF_W.shape[0]), f32),
         jax.ShapeDtypeStruct((b, fc_sr_W.shape[0]), f32)),
        s_acc, den, fc_out_W.T.astype(f32), row(prelu2_a), feat2_last,
        fc_RF_W.T.astype(f32), row(fc_RF_b), fc_sr_W.T.astype(f32))

    return (dec, logits)


# speedup vs baseline: 6.4060x; 1.0085x over previous
import functools

import jax
import jax.numpy as jnp
from jax import lax
from jax.experimental import pallas as pl
from jax.experimental.pallas import tpu as pltpu
from jax.experimental.pallas import tpu_sc as plsc


def _bn_cols(x, g, b):
    m = jnp.mean(x, axis=0, keepdims=True)
    v = jnp.mean((x - m) ** 2, axis=0, keepdims=True)
    return (x - m) * jax.lax.rsqrt(v + 1e-5) * g + b


def _dot(a, b):
    return jnp.dot(a, b, preferred_element_type=jnp.float32)


def _pre_body(feat_ref, g_ref, b_ref, wself_t_ref, fb_ref, s_ref):
    fb = _bn_cols(feat_ref[...], g_ref[...], b_ref[...])
    fb_ref[...] = fb
    s_ref[...] = _dot(fb, wself_t_ref[...])


_GRU_BLK = 1024


def _gru_mega_body(nsteps_ref, countsf_ref, wih_t_ref, bih_ref, whh_t_ref,
                   bhh_ref, xs_ref, out_ref, xbuf_ref, sem_ref):
    blk = _GRU_BLK
    d = out_ref.shape[1]
    out_ref[...] = jnp.zeros(out_ref.shape, out_ref.dtype)
    nsteps = nsteps_ref[0]
    cf = countsf_ref[...]
    wih = wih_t_ref[...]
    whh = whh_t_ref[...]
    bih = bih_ref[...]
    bhh = bhh_ref[...]

    def dma(slot, start):
        return pltpu.make_async_copy(
            xs_ref.at[pl.ds(start, blk)], xbuf_ref.at[slot], sem_ref.at[slot])

    def step(t, R):
        tf = t.astype(jnp.float32)
        K = jnp.sum((cf > tf).astype(jnp.float32)).astype(jnp.int32)
        nblk = (K + blk - 1) // blk
        dma(0, R).start()

        def body(b, _):
            base = b * blk
            slot = jax.lax.rem(b, 2)

            @pl.when(b + 1 < nblk)
            def _():
                dma(jax.lax.rem(b + 1, 2), R + base + blk).start()

            dma(slot, R + base).wait()
            x = xbuf_ref[slot]
            gi = _dot(x.astype(jnp.bfloat16), wih) + bih
            h = out_ref[pl.ds(base, blk), :]
            gh = _dot(h.astype(jnp.bfloat16), whh) + bhh
            ir, iz, inew = gi[:, :d], gi[:, d:2 * d], gi[:, 2 * d:]
            hr, hz, hnn = gh[:, :d], gh[:, d:2 * d], gh[:, 2 * d:]
            r = jax.nn.sigmoid(ir + hr)
            z = jax.nn.sigmoid(iz + hz)
            cand = jnp.tanh(inew + r * hnn)
            hn = (1.0 - z) * cand + z * h
            rowid = base + jax.lax.broadcasted_iota(jnp.int32, (blk, 1), 0)
            out_ref[pl.ds(base, blk), :] = jnp.where(rowid < K, hn, h)
            return 0

        jax.lax.fori_loop(0, nblk, body, 0)
        return R + K

    jax.lax.fori_loop(0, nsteps, step, jnp.int32(0))


def _post_eopa_body(s_ref, h_ref, wneigh_t_ref, a0_ref, feat_ref,
                    g1_ref, b1_ref, wqv_t_ref, qvb_ref, wk_t_ref,
                    feat1_ref, qv_ref, k_ref):
    out0 = s_ref[...] + _dot(h_ref[...], wneigh_t_ref[...])
    a0 = a0_ref[...]
    out0 = jnp.where(out0 > 0, out0, a0 * out0)
    feat1 = jnp.concatenate([out0, feat_ref[...]], axis=1)
    feat1_ref[...] = feat1
    fb1 = _bn_cols(feat1, g1_ref[...], b1_ref[...])
    qv_ref[...] = _dot(fb1, wqv_t_ref[...]) + qvb_ref[...]
    k_ref[...] = _dot(fb1, wk_t_ref[...])


def _feat2_body(rst_ref, a1_ref, feat1_ref, feat2_ref):
    rst = rst_ref[...]
    a1 = a1_ref[...]
    out1 = jnp.where(rst > 0, rst, a1 * rst)
    feat2_ref[...] = jnp.concatenate([out1, feat1_ref[...]], axis=1)


def _bn2_body(feat2_ref, g2_ref, b2_ref, wu_t_ref, fb2_ref, fu_ref,
              mean_ref, rstd_ref):
    x = feat2_ref[...]
    m = jnp.mean(x, axis=0, keepdims=True)
    v = jnp.mean((x - m) ** 2, axis=0, keepdims=True)
    rs = jax.lax.rsqrt(v + 1e-5)
    mean_ref[...] = m
    rstd_ref[...] = rs
    fb2 = (x - m) * rs * g2_ref[...] + b2_ref[...]
    fb2_ref[...] = fb2
    fu_ref[...] = _dot(fb2, wu_t_ref[...])


def _fv_body(f2l_ref, mean_ref, rstd_ref, g2_ref, b2_ref, wv2_t_ref,
             bv2_ref, fv_ref):
    fb2l = ((f2l_ref[...] - mean_ref[...]) * rstd_ref[...] * g2_ref[...]
            + b2_ref[...])
    fv_ref[...] = _dot(fb2l, wv2_t_ref[...]) + bv2_ref[...]


def _read_a_body(fu_ref, seg_ref, fv_ref, we2_ref, e2_ref):
    nb = fu_ref.shape[0]
    nseg = fv_ref.shape[0]
    oh = (seg_ref[...] ==
          jax.lax.broadcasted_iota(jnp.int32, (nb, nseg), 1)).astype(
              jnp.float32)
    fvn = _dot(oh, fv_ref[...])
    sig = jax.nn.sigmoid(fu_ref[...] + fvn)
    e2_ref[...] = jnp.sum(sig * we2_ref[...], axis=1, keepdims=True)


def _read_b_body(e2_ref, mx_ref, seg_ref, fb2_ref, s_ref, den_ref):
    i = pl.program_id(0)

    @pl.when(i == 0)
    def _():
        s_ref[...] = jnp.zeros(s_ref.shape, s_ref.dtype)
        den_ref[...] = jnp.zeros(den_ref.shape, den_ref.dtype)

    nb = e2_ref.shape[0]
    nseg = s_ref.shape[0]
    ex = jnp.exp(e2_ref[...] - mx_ref[0])
    oh = (seg_ref[...] ==
          jax.lax.broadcasted_iota(jnp.int32, (nb, nseg), 1)).astype(
              jnp.float32)
    w = oh * ex
    dn = (((0,), (0,)), ((), ()))
    s_ref[...] += jax.lax.dot_general(
        w, fb2_ref[...], dn, preferred_element_type=jnp.float32)
    den_ref[...] += jax.lax.dot_general(
        w, jnp.ones((nb, den_ref.shape[1]), jnp.float32), dn,
        preferred_element_type=jnp.float32)


def _final_body(s_ref, den_ref, wout_t_ref, a2_ref, feat2l_ref, wrf_t_ref,
                rfb_ref, wsr_t_ref, dec_ref, logits_ref):
    den0 = den_ref[...][:, :1]
    srg_raw = jnp.where(den0 > 0, s_ref[...] / den0, 0.0)
    srg = _dot(srg_raw, wout_t_ref[...])
    a2 = a2_ref[...]
    srg = jnp.where(srg > 0, srg, a2 * srg)
    sr = jnp.concatenate([feat2l_ref[...], srg], axis=1)
    dec_ref[...] = _dot(sr, wrf_t_ref[...]) + rfb_ref[...]
    logits_ref[...] = _dot(sr, wsr_t_ref[...])


_SC_NW = 32
_SC_CH = 128


def _sc_gather_rows(table, idx):
    m = idx.shape[0]
    d = table.shape[1]
    per_w = m // _SC_NW
    nch = per_w // _SC_CH
    idxm = idx.reshape(m // _SC_CH, _SC_CH)
    mesh = plsc.VectorSubcoreMesh(core_axis_name="c", subcore_axis_name="s")

    nbuf = 4 if d <= 128 else 2

    @functools.partial(
        pl.kernel, mesh=mesh,
        out_type=jax.ShapeDtypeStruct((m, d), table.dtype),
        scratch_types=[
            pltpu.VMEM((nch, _SC_CH), jnp.int32),
            pltpu.VMEM((nbuf, _SC_CH, d), table.dtype),
            pltpu.SemaphoreType.DMA((nbuf,)),
            pltpu.SemaphoreType.DMA((nbuf,)),
        ],
    )
    def gather_k(table_hbm, idxm_hbm, out_hbm, idx_v, rows_v, gsem, wsem):
        wid = lax.axis_index("s") * 2 + lax.axis_index("c")
        base = wid * per_w
        pltpu.sync_copy(idxm_hbm.at[pl.ds(wid * nch, nch)], idx_v)
        for j in range(nbuf - 1):
            if j < nch:
                pltpu.async_copy(table_hbm.at[idx_v.at[j]], rows_v.at[j],
                                 gsem.at[j])

        def body(c, _):
            slot = lax.rem(c, nbuf)
            pltpu.make_async_copy(
                table_hbm.at[idx_v.at[c]], rows_v.at[slot],
                gsem.at[slot]).wait()
            pltpu.async_copy(
                rows_v.at[slot], out_hbm.at[pl.ds(base + c * _SC_CH, _SC_CH)],
                wsem.at[slot])

            @pl.when(c + nbuf - 1 < nch)
            def _():
                ns = lax.rem(c + nbuf - 1, nbuf)

                @pl.when(c >= 1)
                def _():
                    pltpu.make_async_copy(
                        rows_v.at[ns], out_hbm.at[pl.ds(base, _SC_CH)],
                        wsem.at[ns]).wait()

                pltpu.async_copy(
                    table_hbm.at[idx_v.at[c + nbuf - 1]], rows_v.at[ns],
                    gsem.at[ns])

            return 0

        lax.fori_loop(0, nch, body, 0, unroll=False)
        for c in range(max(nch - nbuf, 0), nch):
            pltpu.make_async_copy(
                rows_v.at[c % nbuf], out_hbm.at[pl.ds(base, _SC_CH)],
                wsem.at[c % nbuf]).wait()

    return gather_k(table, idxm)


def _sc_build_x(fb0, src_m, rank_m, dst_m, g_tab, nodepos, m_out, dump):
    nch_all, ch = src_m.shape
    m = nch_all * ch
    d = fb0.shape[1]
    per_w = m // _SC_NW
    nch = per_w // ch
    mesh = plsc.VectorSubcoreMesh(core_axis_name="c", subcore_axis_name="s")

    @functools.partial(
        pl.kernel, mesh=mesh,
        out_type=jax.ShapeDtypeStruct((m_out, d), fb0.dtype),
        scratch_types=[
            pltpu.VMEM((nch, ch), jnp.int32),
            pltpu.VMEM((nch, ch), jnp.int32),
            pltpu.VMEM((nch, ch), jnp.int32),
            pltpu.VMEM((2, ch), jnp.int32),
            pltpu.VMEM((2, ch), jnp.int32),
            pltpu.VMEM((2, ch), jnp.int32),
            pltpu.VMEM((2, ch, d), jnp.float32),
            pltpu.SemaphoreType.DMA((2,)),
            pltpu.SemaphoreType.DMA((2,)),
            pltpu.SemaphoreType.DMA((2,)),
            pltpu.SemaphoreType.DMA((2,)),
        ],
    )
    def build_k(fb_hbm, srcm_hbm, rankm_hbm, dstm_hbm, gt_hbm, np_hbm,
                out_hbm, src_v, rank_v, dst_v, gt_v, np_v, pos_v, rows_v,
                gsem, s1sem, s2sem, wsem):
        wid = lax.axis_index("s") * 2 + lax.axis_index("c")
        pltpu.sync_copy(srcm_hbm.at[pl.ds(wid * nch, nch)], src_v)
        pltpu.sync_copy(rankm_hbm.at[pl.ds(wid * nch, nch)], rank_v)
        pltpu.sync_copy(dstm_hbm.at[pl.ds(wid * nch, nch)], dst_v)
        pltpu.async_copy(fb_hbm.at[src_v.at[0]], rows_v.at[0], gsem.at[0])
        pltpu.async_copy(gt_hbm.at[rank_v.at[0]], gt_v.at[0], s1sem.at[0])
        pltpu.async_copy(np_hbm.at[dst_v.at[0]], np_v.at[0], s2sem.at[0])

        def body(c, _):
            slot = lax.rem(c, 2)
            nslot = lax.rem(c + 1, 2)

            @pl.when(c + 1 < nch)
            def _():
                @pl.when(c >= 1)
                def _():
                    pltpu.make_async_copy(rows_v.at[nslot],
                                          out_hbm.at[pos_v.at[nslot]],
                                          wsem.at[nslot]).wait()

                pltpu.async_copy(fb_hbm.at[src_v.at[c + 1]],
                                 rows_v.at[nslot], gsem.at[nslot])
                pltpu.async_copy(gt_hbm.at[rank_v.at[c + 1]],
                                 gt_v.at[nslot], s1sem.at[nslot])
                pltpu.async_copy(np_hbm.at[dst_v.at[c + 1]],
                                 np_v.at[nslot], s2sem.at[nslot])

            pltpu.make_async_copy(gt_hbm.at[rank_v.at[c]], gt_v.at[slot],
                                  s1sem.at[slot]).wait()
            pltpu.make_async_copy(np_hbm.at[dst_v.at[c]], np_v.at[slot],
                                  s2sem.at[slot]).wait()
            for s in (0, 1):
                @pl.when(slot == s)
                def _():
                    for i in range(ch // 16):
                        sl = pl.ds(i * 16, 16)
                        pos_v[s, sl] = jnp.minimum(
                            gt_v[s, sl] + np_v[s, sl], dump)

            pltpu.make_async_copy(fb_hbm.at[src_v.at[c]], rows_v.at[slot],
                                  gsem.at[slot]).wait()
            pltpu.async_copy(rows_v.at[slot], out_hbm.at[pos_v.at[slot]],
                             wsem.at[slot])
            return 0

        lax.fori_loop(0, nch, body, 0, unroll=False)
        for c in range(max(nch - 2, 0), nch):
            pltpu.make_async_copy(rows_v.at[c % 2],
                                  out_hbm.at[pos_v.at[c % 2]],
                                  wsem.at[c % 2]).wait()

    return build_k(fb0, src_m, rank_m, dst_m, g_tab, nodepos)


def _edge_e_body(qvs_ref, kd_ref, w_ref, out_ref):
    d = kd_ref.shape[1]
    sig = jax.nn.sigmoid(qvs_ref[...][:, :d] + kd_ref[...])
    out_ref[...] = jnp.sum(sig * w_ref[...], axis=1, keepdims=True)


def _call(body, out_shapes, *args):
    return pl.pallas_call(
        body,
        out_shape=out_shapes,
    )(*args)


def _seg_softmax(e, seg, num):
    ex = jnp.exp(e - jnp.max(e))
    den = jax.ops.segment_sum(ex, seg, num_segments=num)
    return ex / den[seg]


def kernel(iid, edge_index_mg, edge_index_sg, segment_ids, last_nodes,
           emb_table, bn0_gamma, bn0_beta, gru_Wih, gru_Whh, gru_bih, gru_bhh,
           fc_self_W, fc_neigh_W, prelu0_a, bn1_gamma, bn1_beta,
           fc_q_W, fc_q_b, fc_k_W, fc_v_W, fc_e1_W, prelu1_a,
           bn2_gamma, bn2_beta, fc_u_W, fc_v2_W, fc_v2_b, fc_e2_W, fc_out_W,
           prelu2_a, fc_RF_W, fc_RF_b, fc_sr_W):
    n = iid.shape[0]
    d = emb_table.shape[1]
    b = last_nodes.shape[0]
    f32 = jnp.float32

    def row(x):
        return x.reshape(1, -1).astype(f32)

    feat = emb_table[iid]

    src, dst = edge_index_mg[0], edge_index_mg[1]
    e = src.shape[0]
    order = jnp.argsort(dst)
    src_s = src[order].astype(jnp.int32)
    dst_s = dst[order].astype(jnp.int32)
    counts = jnp.bincount(dst, length=n)
    ar = jnp.arange(e, dtype=jnp.int32)
    is_start = jnp.concatenate(
        [jnp.ones((1,), jnp.bool_), dst_s[1:] != dst_s[:-1]])
    seg_start = jax.lax.cummax(jnp.where(is_start, ar, 0))
    rank_s = ar - seg_start
    maxdeg = counts.max()

    fb0, S = _call(
        _pre_body,
        (jax.ShapeDtypeStruct((n, d), f32),
         jax.ShapeDtypeStruct((n, d), f32)),
        feat, row(bn0_gamma), row(bn0_beta), fc_self_W.T.astype(f32))

    blk = _GRU_BLK
    np_pad = ((n + blk - 1) // blk) * blk
    perm = jnp.argsort(-counts, stable=False)
    nodepos = jnp.zeros((n,), jnp.int32).at[perm].set(
        jnp.arange(n, dtype=jnp.int32))
    counts_p = jnp.zeros((np_pad,), jnp.int32).at[:n].set(counts[perm])
    countsf = counts_p.astype(f32).reshape(np_pad // 128, 128)

    hist = jnp.bincount(counts, length=e + 1)
    k_of_t = n - jnp.cumsum(hist)
    g_tab = jnp.concatenate(
        [jnp.zeros((1,), jnp.int32),
         jnp.cumsum(k_of_t).astype(jnp.int32)])
    m_e = ((e + 4095) // 4096) * 4096
    m_g = ((e + blk + 8 + 4095) // 4096) * 4096 + 4096

    def padm(x, fill):
        return jnp.full((m_e,), fill, jnp.int32).at[:e].set(x).reshape(
            m_e // _SC_CH, _SC_CH)

    x_stream = _sc_build_x(
        fb0, padm(src_s, 0), padm(rank_s, e + 1), padm(dst_s, 0),
        g_tab, nodepos, m_g, m_g - 1)

    nsteps = maxdeg.astype(jnp.int32).reshape(1)
    h_p = pl.pallas_call(
        _gru_mega_body,
        out_shape=jax.ShapeDtypeStruct((np_pad, d), f32),
        in_specs=[
            pl.BlockSpec(memory_space=pltpu.SMEM),
            pl.BlockSpec(memory_space=pltpu.MemorySpace.VMEM),
            pl.BlockSpec(memory_space=pltpu.MemorySpace.VMEM),
            pl.BlockSpec(memory_space=pltpu.MemorySpace.VMEM),
            pl.BlockSpec(memory_space=pltpu.MemorySpace.VMEM),
            pl.BlockSpec(memory_space=pltpu.MemorySpace.VMEM),
            pl.BlockSpec(memory_space=pltpu.MemorySpace.HBM),
        ],
        out_specs=pl.BlockSpec(memory_space=pltpu.MemorySpace.VMEM),
        scratch_shapes=[
            pltpu.VMEM((2, blk, d), f32),
            pltpu.SemaphoreType.DMA((2,)),
        ],
    )(nsteps, countsf, gru_Wih.T.astype(jnp.bfloat16), row(gru_bih),
      gru_Whh.T.astype(jnp.bfloat16), row(gru_bhh), x_stream)
    m_n = ((n + 32767) // 32768) * 32768
    nodepos_p = jnp.zeros((m_n,), jnp.int32).at[:n].set(nodepos)
    neigh = _sc_gather_rows(h_p, nodepos_p)[:n]

    wqv_t = jnp.concatenate(
        [fc_q_W.T.astype(f32), fc_v_W.T.astype(f32)], axis=1)
    qvb = jnp.concatenate(
        [fc_q_b.astype(f32), jnp.zeros((d,), f32)]).reshape(1, 2 * d)
    feat1, qv, k = _call(
        _post_eopa_body,
        (jax.ShapeDtypeStruct((n, 2 * d), f32),
         jax.ShapeDtypeStruct((n, 2 * d), f32),
         jax.ShapeDtypeStruct((n, d), f32)),
        S, neigh, fc_neigh_W.T.astype(f32), row(prelu0_a), feat,
        row(bn1_gamma), row(bn1_beta), wqv_t, qvb, fc_k_W.T.astype(f32))

    src2 = edge_index_sg[0].astype(jnp.int32)
    dst2 = edge_index_sg[1].astype(jnp.int32)
    src2_p = jnp.zeros((m_e,), jnp.int32).at[:e].set(src2)
    dst2_p = jnp.zeros((m_e,), jnp.int32).at[:e].set(dst2)
    qvs = _sc_gather_rows(qv, src2_p)
    kd = _sc_gather_rows(k, dst2_p)

    eblk = 8192
    e_att = pl.pallas_call(
        _edge_e_body,
        grid=(m_e // eblk,),
        in_specs=[
            pl.BlockSpec((eblk, 2 * d), lambda i: (i, 0)),
            pl.BlockSpec((eblk, d), lambda i: (i, 0)),
            pl.BlockSpec((1, d), lambda i: (0, 0)),
        ],
        out_specs=pl.BlockSpec((eblk, 1), lambda i: (i, 0)),
        out_shape=jax.ShapeDtypeStruct((m_e, 1), f32),
    )(qvs, kd, fc_e1_W.astype(f32))
    e_att = e_att[:e, 0]
    ex = jnp.exp(e_att - jnp.max(e_att))
    den = jax.ops.segment_sum(ex, dst2, num_segments=n)
    rst_u = jax.ops.segment_sum(qvs[:e, d:] * ex[:, None], dst2,
                                num_segments=n)
    rst = jnp.where(den[:, None] > 0, rst_u / den[:, None], 0.0)

    feat2 = _call(
        _feat2_body,
        jax.ShapeDtypeStruct((n, 3 * d), f32),
        rst, row(prelu1_a), feat1)
    fb2, fu, mean2, rstd2 = _call(
        _bn2_body,
        (jax.ShapeDtypeStruct((n, 3 * d), f32),
         jax.ShapeDtypeStruct((n, d), f32),
         jax.ShapeDtypeStruct((1, 3 * d), f32),
         jax.ShapeDtypeStruct((1, 3 * d), f32)),
        feat2, row(bn2_gamma), row(bn2_beta), fc_u_W.T.astype(f32))

    feat2_last = feat2[last_nodes]
    fv = _call(
        _fv_body, jax.ShapeDtypeStruct((b, d), f32),
        feat2_last, mean2, rstd2, row(bn2_gamma), row(bn2_beta),
        fc_v2_W.T.astype(f32), row(fc_v2_b))

    segc = segment_ids.astype(jnp.int32).reshape(n, 1)
    rblk = 2000
    e2 = pl.pallas_call(
        _read_a_body,
        grid=(n // rblk,),
        in_specs=[
            pl.BlockSpec((rblk, d), lambda i: (i, 0)),
            pl.BlockSpec((rblk, 1), lambda i: (i, 0)),
            pl.BlockSpec((b, d), lambda i: (0, 0)),
            pl.BlockSpec((1, d), lambda i: (0, 0)),
        ],
        out_specs=pl.BlockSpec((rblk, 1), lambda i: (i, 0)),
        out_shape=jax.ShapeDtypeStruct((n, 1), f32),
    )(fu, segc, fv, fc_e2_W.astype(f32))

    mx = jnp.max(e2).reshape(1)
    s_acc, den = pl.pallas_call(
        _read_b_body,
        grid=(n // rblk,),
        in_specs=[
            pl.BlockSpec((rblk, 1), lambda i: (i, 0)),
            pl.BlockSpec(memory_space=pltpu.SMEM),
            pl.BlockSpec((rblk, 1), lambda i: (i, 0)),
            pl.BlockSpec((rblk, 3 * d), lambda i: (i, 0)),
        ],
        out_specs=(pl.BlockSpec((b, 3 * d), lambda i: (0, 0)),
                   pl.BlockSpec((b, 8), lambda i: (0, 0))),
        out_shape=(jax.ShapeDtypeStruct((b, 3 * d), f32),
                   jax.ShapeDtypeStruct((b, 8), f32)),
    )(e2, mx, segc, fb2)

    dec, logits = _call(
        _final_body,
        (jax.ShapeDtypeStruct((b, fc_RF_W.shape[0]), f32),
         jax.ShapeDtypeStruct((b, fc_sr_W.shape[0]), f32)),
        s_acc, den, fc_out_W.T.astype(f32), row(prelu2_a), feat2_last,
        fc_RF_W.T.astype(f32), row(fc_RF_b), fc_sr_W.T.astype(f32))

    return (dec, logits)


# speedup vs baseline: 6.9679x; 1.0877x over previous
import functools

import jax
import jax.numpy as jnp
from jax import lax
from jax.experimental import pallas as pl
from jax.experimental.pallas import tpu as pltpu
from jax.experimental.pallas import tpu_sc as plsc


def _bn_cols(x, g, b):
    m = jnp.mean(x, axis=0, keepdims=True)
    v = jnp.mean((x - m) ** 2, axis=0, keepdims=True)
    return (x - m) * jax.lax.rsqrt(v + 1e-5) * g + b


def _dot(a, b):
    return jnp.dot(a, b, preferred_element_type=jnp.float32)


def _pre_body(feat_ref, g_ref, b_ref, wself_t_ref, fb_ref, s_ref):
    fb = _bn_cols(feat_ref[...], g_ref[...], b_ref[...])
    fb_ref[...] = fb
    s_ref[...] = _dot(fb, wself_t_ref[...])


_GRU_BLK = 1024


def _gru_mega_body(nsteps_ref, countsf_ref, wih_t_ref, bih_ref, whh_t_ref,
                   bhh_ref, xs_ref, out_ref, xbuf_ref, sem_ref):
    blk = _GRU_BLK
    d = out_ref.shape[1]
    out_ref[...] = jnp.zeros(out_ref.shape, out_ref.dtype)
    nsteps = nsteps_ref[0]
    cf = countsf_ref[...]
    wih = wih_t_ref[...]
    whh = whh_t_ref[...]
    bih = bih_ref[...]
    bhh = bhh_ref[...]

    def dma(slot, start):
        return pltpu.make_async_copy(
            xs_ref.at[pl.ds(start, blk)], xbuf_ref.at[slot], sem_ref.at[slot])

    def step(t, R):
        tf = t.astype(jnp.float32)
        K = jnp.sum((cf > tf).astype(jnp.float32)).astype(jnp.int32)
        nblk = (K + blk - 1) // blk
        dma(0, R).start()

        def body(b, _):
            base = b * blk
            slot = jax.lax.rem(b, 2)

            @pl.when(b + 1 < nblk)
            def _():
                dma(jax.lax.rem(b + 1, 2), R + base + blk).start()

            dma(slot, R + base).wait()
            x = xbuf_ref[slot]
            gi = _dot(x.astype(jnp.bfloat16), wih) + bih
            h = out_ref[pl.ds(base, blk), :]
            gh = _dot(h.astype(jnp.bfloat16), whh) + bhh
            ir, iz, inew = gi[:, :d], gi[:, d:2 * d], gi[:, 2 * d:]
            hr, hz, hnn = gh[:, :d], gh[:, d:2 * d], gh[:, 2 * d:]
            r = jax.nn.sigmoid(ir + hr)
            z = jax.nn.sigmoid(iz + hz)
            cand = jnp.tanh(inew + r * hnn)
            hn = (1.0 - z) * cand + z * h
            rowid = base + jax.lax.broadcasted_iota(jnp.int32, (blk, 1), 0)
            out_ref[pl.ds(base, blk), :] = jnp.where(rowid < K, hn, h)
            return 0

        jax.lax.fori_loop(0, nblk, body, 0)
        return R + K

    jax.lax.fori_loop(0, nsteps, step, jnp.int32(0))


def _post_eopa_body(s_ref, h_ref, wneigh_t_ref, a0_ref, feat_ref,
                    g1_ref, b1_ref, wqv_t_ref, qvb_ref, wk_t_ref,
                    feat1_ref, qv_ref, k_ref):
    out0 = s_ref[...] + _dot(h_ref[...], wneigh_t_ref[...])
    a0 = a0_ref[...]
    out0 = jnp.where(out0 > 0, out0, a0 * out0)
    feat1 = jnp.concatenate([out0, feat_ref[...]], axis=1)
    feat1_ref[...] = feat1
    fb1 = _bn_cols(feat1, g1_ref[...], b1_ref[...])
    qv_ref[...] = _dot(fb1, wqv_t_ref[...]) + qvb_ref[...]
    k_ref[...] = _dot(fb1, wk_t_ref[...])


def _feat2_body(rst_ref, a1_ref, feat1_ref, feat2_ref):
    rst = rst_ref[...]
    a1 = a1_ref[...]
    out1 = jnp.where(rst > 0, rst, a1 * rst)
    feat2_ref[...] = jnp.concatenate([out1, feat1_ref[...]], axis=1)


def _bn2_body(feat2_ref, g2_ref, b2_ref, wu_t_ref, fb2_ref, fu_ref,
              mean_ref, rstd_ref):
    x = feat2_ref[...]
    m = jnp.mean(x, axis=0, keepdims=True)
    v = jnp.mean((x - m) ** 2, axis=0, keepdims=True)
    rs = jax.lax.rsqrt(v + 1e-5)
    mean_ref[...] = m
    rstd_ref[...] = rs
    fb2 = (x - m) * rs * g2_ref[...] + b2_ref[...]
    fb2_ref[...] = fb2
    fu_ref[...] = _dot(fb2, wu_t_ref[...])


def _fv_body(f2l_ref, mean_ref, rstd_ref, g2_ref, b2_ref, wv2_t_ref,
             bv2_ref, fv_ref):
    fb2l = ((f2l_ref[...] - mean_ref[...]) * rstd_ref[...] * g2_ref[...]
            + b2_ref[...])
    fv_ref[...] = _dot(fb2l, wv2_t_ref[...]) + bv2_ref[...]


def _read_a_body(fu_ref, seg_ref, fv_ref, we2_ref, e2_ref):
    nb = fu_ref.shape[0]
    nseg = fv_ref.shape[0]
    oh = (seg_ref[...] ==
          jax.lax.broadcasted_iota(jnp.int32, (nb, nseg), 1)).astype(
              jnp.float32)
    fvn = _dot(oh, fv_ref[...])
    sig = jax.nn.sigmoid(fu_ref[...] + fvn)
    e2_ref[...] = jnp.sum(sig * we2_ref[...], axis=1, keepdims=True)


def _read_b_body(e2_ref, mx_ref, seg_ref, fb2_ref, s_ref, den_ref):
    i = pl.program_id(0)

    @pl.when(i == 0)
    def _():
        s_ref[...] = jnp.zeros(s_ref.shape, s_ref.dtype)
        den_ref[...] = jnp.zeros(den_ref.shape, den_ref.dtype)

    nb = e2_ref.shape[0]
    nseg = s_ref.shape[0]
    ex = jnp.exp(e2_ref[...] - mx_ref[0])
    oh = (seg_ref[...] ==
          jax.lax.broadcasted_iota(jnp.int32, (nb, nseg), 1)).astype(
              jnp.float32)
    w = oh * ex
    dn = (((0,), (0,)), ((), ()))
    s_ref[...] += jax.lax.dot_general(
        w, fb2_ref[...], dn, preferred_element_type=jnp.float32)
    den_ref[...] += jax.lax.dot_general(
        w, jnp.ones((nb, den_ref.shape[1]), jnp.float32), dn,
        preferred_element_type=jnp.float32)


def _final_body(s_ref, den_ref, wout_t_ref, a2_ref, feat2l_ref, wrf_t_ref,
                rfb_ref, wsr_t_ref, dec_ref, logits_ref):
    den0 = den_ref[...][:, :1]
    srg_raw = jnp.where(den0 > 0, s_ref[...] / den0, 0.0)
    srg = _dot(srg_raw, wout_t_ref[...])
    a2 = a2_ref[...]
    srg = jnp.where(srg > 0, srg, a2 * srg)
    sr = jnp.concatenate([feat2l_ref[...], srg], axis=1)
    dec_ref[...] = _dot(sr, wrf_t_ref[...]) + rfb_ref[...]
    logits_ref[...] = _dot(sr, wsr_t_ref[...])


_SC_NW = 32
_SC_CH = 128


def _sc_gather_rows(table, idx):
    m = idx.shape[0]
    d = table.shape[1]
    per_w = m // _SC_NW
    nch = per_w // _SC_CH
    idxm = idx.reshape(m // _SC_CH, _SC_CH)
    mesh = plsc.VectorSubcoreMesh(core_axis_name="c", subcore_axis_name="s")

    nbuf = 4 if d <= 128 else 2

    @functools.partial(
        pl.kernel, mesh=mesh,
        out_type=jax.ShapeDtypeStruct((m, d), table.dtype),
        scratch_types=[
            pltpu.VMEM((nch, _SC_CH), jnp.int32),
            pltpu.VMEM((nbuf, _SC_CH, d), table.dtype),
            pltpu.SemaphoreType.DMA((nbuf,)),
            pltpu.SemaphoreType.DMA((nbuf,)),
        ],
    )
    def gather_k(table_hbm, idxm_hbm, out_hbm, idx_v, rows_v, gsem, wsem):
        wid = lax.axis_index("s") * 2 + lax.axis_index("c")
        base = wid * per_w
        pltpu.sync_copy(idxm_hbm.at[pl.ds(wid * nch, nch)], idx_v)
        for j in range(nbuf - 1):
            if j < nch:
                pltpu.async_copy(table_hbm.at[idx_v.at[j]], rows_v.at[j],
                                 gsem.at[j])

        def body(c, _):
            slot = lax.rem(c, nbuf)
            pltpu.make_async_copy(
                table_hbm.at[idx_v.at[c]], rows_v.at[slot],
                gsem.at[slot]).wait()
            pltpu.async_copy(
                rows_v.at[slot], out_hbm.at[pl.ds(base + c * _SC_CH, _SC_CH)],
                wsem.at[slot])

            @pl.when(c + nbuf - 1 < nch)
            def _():
                ns = lax.rem(c + nbuf - 1, nbuf)

                @pl.when(c >= 1)
                def _():
                    pltpu.make_async_copy(
                        rows_v.at[ns], out_hbm.at[pl.ds(base, _SC_CH)],
                        wsem.at[ns]).wait()

                pltpu.async_copy(
                    table_hbm.at[idx_v.at[c + nbuf - 1]], rows_v.at[ns],
                    gsem.at[ns])

            return 0

        lax.fori_loop(0, nch, body, 0, unroll=False)
        for c in range(max(nch - nbuf, 0), nch):
            pltpu.make_async_copy(
                rows_v.at[c % nbuf], out_hbm.at[pl.ds(base, _SC_CH)],
                wsem.at[c % nbuf]).wait()

    return gather_k(table, idxm)


def _sc_build_x(fb0, src_m, rank_m, dst_m, g_tab, nodepos, m_out, dump):
    nch_all, ch = src_m.shape
    m = nch_all * ch
    d = fb0.shape[1]
    per_w = m // _SC_NW
    nch = per_w // ch
    mesh = plsc.VectorSubcoreMesh(core_axis_name="c", subcore_axis_name="s")

    @functools.partial(
        pl.kernel, mesh=mesh,
        out_type=jax.ShapeDtypeStruct((m_out, d), fb0.dtype),
        scratch_types=[
            pltpu.VMEM((nch, ch), jnp.int32),
            pltpu.VMEM((nch, ch), jnp.int32),
            pltpu.VMEM((nch, ch), jnp.int32),
            pltpu.VMEM((2, ch), jnp.int32),
            pltpu.VMEM((2, ch), jnp.int32),
            pltpu.VMEM((2, ch), jnp.int32),
            pltpu.VMEM((2, ch, d), jnp.float32),
            pltpu.SemaphoreType.DMA((2,)),
            pltpu.SemaphoreType.DMA((2,)),
            pltpu.SemaphoreType.DMA((2,)),
            pltpu.SemaphoreType.DMA((2,)),
        ],
    )
    def build_k(fb_hbm, srcm_hbm, rankm_hbm, dstm_hbm, gt_hbm, np_hbm,
                out_hbm, src_v, rank_v, dst_v, gt_v, np_v, pos_v,
                rows_v, gsem, s1sem, s2sem, wsem):
        wid = lax.axis_index("s") * 2 + lax.axis_index("c")
        pltpu.sync_copy(srcm_hbm.at[pl.ds(wid * nch, nch)], src_v)
        pltpu.sync_copy(rankm_hbm.at[pl.ds(wid * nch, nch)], rank_v)
        pltpu.sync_copy(dstm_hbm.at[pl.ds(wid * nch, nch)], dst_v)
        pltpu.async_copy(fb_hbm.at[src_v.at[0]], rows_v.at[0], gsem.at[0])
        pltpu.async_copy(gt_hbm.at[rank_v.at[0]], gt_v.at[0], s1sem.at[0])
        pltpu.async_copy(np_hbm.at[dst_v.at[0]], np_v.at[0], s2sem.at[0])

        def body(c, _):
            slot = lax.rem(c, 2)
            nslot = lax.rem(c + 1, 2)

            @pl.when(c + 1 < nch)
            def _():
                @pl.when(c >= 1)
                def _():
                    pltpu.make_async_copy(rows_v.at[nslot],
                                          out_hbm.at[pos_v.at[nslot]],
                                          wsem.at[nslot]).wait()

                pltpu.async_copy(fb_hbm.at[src_v.at[c + 1]],
                                 rows_v.at[nslot], gsem.at[nslot])
                pltpu.async_copy(gt_hbm.at[rank_v.at[c + 1]],
                                 gt_v.at[nslot], s1sem.at[nslot])
                pltpu.async_copy(np_hbm.at[dst_v.at[c + 1]],
                                 np_v.at[nslot], s2sem.at[nslot])

            pltpu.make_async_copy(gt_hbm.at[rank_v.at[c]], gt_v.at[slot],
                                  s1sem.at[slot]).wait()
            pltpu.make_async_copy(np_hbm.at[dst_v.at[c]], np_v.at[slot],
                                  s2sem.at[slot]).wait()
            for s in (0, 1):
                @pl.when(slot == s)
                def _():
                    for i in range(ch // 16):
                        sl = pl.ds(i * 16, 16)
                        pos_v[s, sl] = jnp.minimum(
                            gt_v[s, sl] + np_v[s, sl], dump)

            pltpu.make_async_copy(fb_hbm.at[src_v.at[c]], rows_v.at[slot],
                                  gsem.at[slot]).wait()
            pltpu.async_copy(rows_v.at[slot], out_hbm.at[pos_v.at[slot]],
                             wsem.at[slot])
            return 0

        lax.fori_loop(0, nch, body, 0, unroll=False)
        for c in range(max(nch - 2, 0), nch):
            pltpu.make_async_copy(rows_v.at[c % 2],
                                  out_hbm.at[pos_v.at[c % 2]],
                                  wsem.at[c % 2]).wait()

    return build_k(fb0, src_m, rank_m, dst_m, g_tab, nodepos)


def _sc_scatter_add(rows, idx_m, zeros_init):
    m, dext = rows.shape
    nch_all, ch = idx_m.shape
    nseg = zeros_init.shape[0]
    per_w = m // _SC_NW
    nch = per_w // ch
    mesh = plsc.VectorSubcoreMesh(core_axis_name="c", subcore_axis_name="s")

    @functools.partial(
        pl.kernel, mesh=mesh,
        out_type=jax.ShapeDtypeStruct((2, nseg, dext), jnp.float32),
        scratch_types=[
            pltpu.VMEM((nch, ch), jnp.int32),
            pltpu.VMEM((2, ch, dext), jnp.float32),
            pltpu.VMEM_SHARED((nseg, dext), jnp.float32),
            pltpu.SemaphoreType.DMA((2,)),
            pltpu.SemaphoreType.DMA((2,)),
        ],
    )
    def scat_k(rows_hbm, idxm_hbm, zeros_hbm, out_hbm, idx_v, rows_v,
               acc_sh, lsem, ssem):
        cid = lax.axis_index("c")
        sid = lax.axis_index("s")
        wid = sid * 2 + cid

        @pl.when(sid == 0)
        def _():
            pltpu.sync_copy(zeros_hbm, acc_sh)

        plsc.subcore_barrier()
        pltpu.sync_copy(idxm_hbm.at[pl.ds(wid * nch, nch)], idx_v)
        base = wid * per_w
        pltpu.async_copy(rows_hbm.at[pl.ds(base, ch)], rows_v.at[0],
                         lsem.at[0])

        def body(c, _):
            slot = lax.rem(c, 2)
            nslot = lax.rem(c + 1, 2)

            @pl.when(c + 1 < nch)
            def _():
                @pl.when(c >= 1)
                def _():
                    pltpu.make_async_copy(
                        rows_v.at[nslot], acc_sh.at[idx_v.at[c - 1]],
                        ssem.at[nslot]).wait()

                pltpu.async_copy(
                    rows_hbm.at[pl.ds(base + (c + 1) * ch, ch)],
                    rows_v.at[nslot], lsem.at[nslot])

            pltpu.make_async_copy(
                rows_hbm.at[pl.ds(base, ch)], rows_v.at[slot],
                lsem.at[slot]).wait()
            pltpu.async_copy(rows_v.at[slot], acc_sh.at[idx_v.at[c]],
                             ssem.at[slot], add=True)
            return 0

        lax.fori_loop(0, nch, body, 0, unroll=False)
        for c in range(max(nch - 2, 0), nch):
            pltpu.make_async_copy(rows_v.at[c % 2],
                                  acc_sh.at[idx_v.at[c]],
                                  ssem.at[c % 2]).wait()
        plsc.subcore_barrier()

        @pl.when(sid == 0)
        def _():
            pltpu.sync_copy(acc_sh, out_hbm.at[cid])

    return scat_k(rows, idx_m, zeros_init)


def _edge_e_body(qvs_ref, kd_ref, w_ref, out_ref):
    d = kd_ref.shape[1]
    sig = jax.nn.sigmoid(qvs_ref[...][:, :d] + kd_ref[...])
    out_ref[...] = jnp.sum(sig * w_ref[...], axis=1, keepdims=True)


def _call(body, out_shapes, *args):
    return pl.pallas_call(
        body,
        out_shape=out_shapes,
    )(*args)


def _seg_softmax(e, seg, num):
    ex = jnp.exp(e - jnp.max(e))
    den = jax.ops.segment_sum(ex, seg, num_segments=num)
    return ex / den[seg]


def kernel(iid, edge_index_mg, edge_index_sg, segment_ids, last_nodes,
           emb_table, bn0_gamma, bn0_beta, gru_Wih, gru_Whh, gru_bih, gru_bhh,
           fc_self_W, fc_neigh_W, prelu0_a, bn1_gamma, bn1_beta,
           fc_q_W, fc_q_b, fc_k_W, fc_v_W, fc_e1_W, prelu1_a,
           bn2_gamma, bn2_beta, fc_u_W, fc_v2_W, fc_v2_b, fc_e2_W, fc_out_W,
           prelu2_a, fc_RF_W, fc_RF_b, fc_sr_W):
    n = iid.shape[0]
    d = emb_table.shape[1]
    b = last_nodes.shape[0]
    f32 = jnp.float32

    def row(x):
        return x.reshape(1, -1).astype(f32)

    feat = emb_table[iid]

    src, dst = edge_index_mg[0], edge_index_mg[1]
    e = src.shape[0]
    order = jnp.argsort(dst)
    src_s = src[order].astype(jnp.int32)
    dst_s = dst[order].astype(jnp.int32)
    counts = jnp.bincount(dst, length=n)
    ar = jnp.arange(e, dtype=jnp.int32)
    is_start = jnp.concatenate(
        [jnp.ones((1,), jnp.bool_), dst_s[1:] != dst_s[:-1]])
    seg_start = jax.lax.cummax(jnp.where(is_start, ar, 0))
    rank_s = ar - seg_start
    maxdeg = counts.max()

    fb0, S = _call(
        _pre_body,
        (jax.ShapeDtypeStruct((n, d), f32),
         jax.ShapeDtypeStruct((n, d), f32)),
        feat, row(bn0_gamma), row(bn0_beta), fc_self_W.T.astype(f32))

    blk = _GRU_BLK
    np_pad = ((n + blk - 1) // blk) * blk
    perm = jnp.argsort(-counts, stable=False)
    nodepos = jnp.zeros((n,), jnp.int32).at[perm].set(
        jnp.arange(n, dtype=jnp.int32))
    counts_p = jnp.zeros((np_pad,), jnp.int32).at[:n].set(counts[perm])
    countsf = counts_p.astype(f32).reshape(np_pad // 128, 128)

    hist = jnp.bincount(counts, length=e + 1)
    k_of_t = n - jnp.cumsum(hist)
    g_tab = jnp.concatenate(
        [jnp.zeros((1,), jnp.int32),
         jnp.cumsum(k_of_t).astype(jnp.int32)])
    m_e = ((e + 4095) // 4096) * 4096
    m_g = ((e + blk + 8 + 4095) // 4096) * 4096 + 4096

    def padm(x, fill):
        return jnp.full((m_e,), fill, jnp.int32).at[:e].set(x).reshape(
            m_e // _SC_CH, _SC_CH)

    x_stream = _sc_build_x(
        fb0, padm(src_s, 0), padm(rank_s, e + 1), padm(dst_s, 0),
        g_tab, nodepos, m_g, m_g - 1)

    nsteps = maxdeg.astype(jnp.int32).reshape(1)
    h_p = pl.pallas_call(
        _gru_mega_body,
        out_shape=jax.ShapeDtypeStruct((np_pad, d), f32),
        in_specs=[
            pl.BlockSpec(memory_space=pltpu.SMEM),
            pl.BlockSpec(memory_space=pltpu.MemorySpace.VMEM),
            pl.BlockSpec(memory_space=pltpu.MemorySpace.VMEM),
            pl.BlockSpec(memory_space=pltpu.MemorySpace.VMEM),
            pl.BlockSpec(memory_space=pltpu.MemorySpace.VMEM),
            pl.BlockSpec(memory_space=pltpu.MemorySpace.VMEM),
            pl.BlockSpec(memory_space=pltpu.MemorySpace.HBM),
        ],
        out_specs=pl.BlockSpec(memory_space=pltpu.MemorySpace.VMEM),
        scratch_shapes=[
            pltpu.VMEM((2, blk, d), f32),
            pltpu.SemaphoreType.DMA((2,)),
        ],
    )(nsteps, countsf, gru_Wih.T.astype(jnp.bfloat16), row(gru_bih),
      gru_Whh.T.astype(jnp.bfloat16), row(gru_bhh), x_stream)
    m_n = ((n + 32767) // 32768) * 32768
    nodepos_p = jnp.zeros((m_n,), jnp.int32).at[:n].set(nodepos)
    neigh = _sc_gather_rows(h_p, nodepos_p)[:n]

    wqv_t = jnp.concatenate(
        [fc_q_W.T.astype(f32), fc_v_W.T.astype(f32)], axis=1)
    qvb = jnp.concatenate(
        [fc_q_b.astype(f32), jnp.zeros((d,), f32)]).reshape(1, 2 * d)
    feat1, qv, k = _call(
        _post_eopa_body,
        (jax.ShapeDtypeStruct((n, 2 * d), f32),
         jax.ShapeDtypeStruct((n, 2 * d), f32),
         jax.ShapeDtypeStruct((n, d), f32)),
        S, neigh, fc_neigh_W.T.astype(f32), row(prelu0_a), feat,
        row(bn1_gamma), row(bn1_beta), wqv_t, qvb, fc_k_W.T.astype(f32))

    src2 = edge_index_sg[0].astype(jnp.int32)
    dst2 = edge_index_sg[1].astype(jnp.int32)
    src2_p = jnp.zeros((m_e,), jnp.int32).at[:e].set(src2)
    dst2_p = jnp.zeros((m_e,), jnp.int32).at[:e].set(dst2)
    qvs = _sc_gather_rows(qv, src2_p)
    kd = _sc_gather_rows(k, dst2_p)

    eblk = 8192
    e_att = pl.pallas_call(
        _edge_e_body,
        grid=(m_e // eblk,),
        in_specs=[
            pl.BlockSpec((eblk, 2 * d), lambda i: (i, 0)),
            pl.BlockSpec((eblk, d), lambda i: (i, 0)),
            pl.BlockSpec((1, d), lambda i: (0, 0)),
        ],
        out_specs=pl.BlockSpec((eblk, 1), lambda i: (i, 0)),
        out_shape=jax.ShapeDtypeStruct((m_e, 1), f32),
    )(qvs, kd, fc_e1_W.astype(f32))
    e_att = e_att[:e, 0]
    ex = jnp.exp(e_att - jnp.max(e_att))
    wrows = jnp.zeros((m_e, d), f32).at[:e].set(qvs[:e, d:] * ex[:, None])
    acc = _sc_scatter_add(
        wrows, dst2_p.reshape(m_e // _SC_CH, _SC_CH),
        jnp.zeros((n, d), f32))
    rst_u = acc[0] + acc[1]
    den = jax.ops.segment_sum(ex, dst2, num_segments=n)
    rst = jnp.where(den[:, None] > 0, rst_u / den[:, None], 0.0)

    feat2 = _call(
        _feat2_body,
        jax.ShapeDtypeStruct((n, 3 * d), f32),
        rst, row(prelu1_a), feat1)
    fb2, fu, mean2, rstd2 = _call(
        _bn2_body,
        (jax.ShapeDtypeStruct((n, 3 * d), f32),
         jax.ShapeDtypeStruct((n, d), f32),
         jax.ShapeDtypeStruct((1, 3 * d), f32),
         jax.ShapeDtypeStruct((1, 3 * d), f32)),
        feat2, row(bn2_gamma), row(bn2_beta), fc_u_W.T.astype(f32))

    feat2_last = feat2[last_nodes]
    fv = _call(
        _fv_body, jax.ShapeDtypeStruct((b, d), f32),
        feat2_last, mean2, rstd2, row(bn2_gamma), row(bn2_beta),
        fc_v2_W.T.astype(f32), row(fc_v2_b))

    segc = segment_ids.astype(jnp.int32).reshape(n, 1)
    rblk = 2000
    e2 = pl.pallas_call(
        _read_a_body,
        grid=(n // rblk,),
        in_specs=[
            pl.BlockSpec((rblk, d), lambda i: (i, 0)),
            pl.BlockSpec((rblk, 1), lambda i: (i, 0)),
            pl.BlockSpec((b, d), lambda i: (0, 0)),
            pl.BlockSpec((1, d), lambda i: (0, 0)),
        ],
        out_specs=pl.BlockSpec((rblk, 1), lambda i: (i, 0)),
        out_shape=jax.ShapeDtypeStruct((n, 1), f32),
    )(fu, segc, fv, fc_e2_W.astype(f32))

    mx = jnp.max(e2).reshape(1)
    s_acc, den = pl.pallas_call(
        _read_b_body,
        grid=(n // rblk,),
        in_specs=[
            pl.BlockSpec((rblk, 1), lambda i: (i, 0)),
            pl.BlockSpec(memory_space=pltpu.SMEM),
            pl.BlockSpec((rblk, 1), lambda i: (i, 0)),
            pl.BlockSpec((rblk, 3 * d), lambda i: (i, 0)),
        ],
        out_specs=(pl.BlockSpec((b, 3 * d), lambda i: (0, 0)),
                   pl.BlockSpec((b, 8), lambda i: (0, 0))),
        out_shape=(jax.ShapeDtypeStruct((b, 3 * d), f32),
                   jax.ShapeDtypeStruct((b, 8), f32)),
    )(e2, mx, segc, fb2)

    dec, logits = _call(
        _final_body,
        (jax.ShapeDtypeStruct((b, fc_RF_W.shape[0]), f32),
         jax.ShapeDtypeStruct((b, fc_sr_W.shape[0]), f32)),
        s_acc, den, fc_out_W.T.astype(f32), row(prelu2_a), feat2_last,
        fc_RF_W.T.astype(f32), row(fc_RF_b), fc_sr_W.T.astype(f32))

    return (dec, logits)


# speedup vs baseline: 9.3032x; 1.3352x over previous
import functools

import jax
import jax.numpy as jnp
from jax import lax
from jax.experimental import pallas as pl
from jax.experimental.pallas import tpu as pltpu
from jax.experimental.pallas import tpu_sc as plsc


def _bn_cols(x, g, b):
    m = jnp.mean(x, axis=0, keepdims=True)
    v = jnp.mean((x - m) ** 2, axis=0, keepdims=True)
    return (x - m) * jax.lax.rsqrt(v + 1e-5) * g + b


def _dot(a, b):
    return jnp.dot(a, b, preferred_element_type=jnp.float32)


def _pre_body(feat_ref, g_ref, b_ref, wself_t_ref, fb_ref, s_ref):
    fb = _bn_cols(feat_ref[...], g_ref[...], b_ref[...])
    fb_ref[...] = fb
    s_ref[...] = _dot(fb, wself_t_ref[...])


_GRU_BLK = 1024


def _gru_mega_body(nsteps_ref, countsf_ref, wih_t_ref, bih_ref, whh_t_ref,
                   bhh_ref, xs_ref, out_ref, xbuf_ref, sem_ref):
    blk = _GRU_BLK
    d = out_ref.shape[1]
    out_ref[...] = jnp.zeros(out_ref.shape, out_ref.dtype)
    nsteps = nsteps_ref[0]
    cf = countsf_ref[...]
    wih = wih_t_ref[...]
    whh = whh_t_ref[...]
    bih = bih_ref[...]
    bhh = bhh_ref[...]

    def dma(slot, start):
        return pltpu.make_async_copy(
            xs_ref.at[pl.ds(start, blk)], xbuf_ref.at[slot], sem_ref.at[slot])

    def step(t, R):
        tf = t.astype(jnp.float32)
        K = jnp.sum((cf > tf).astype(jnp.float32)).astype(jnp.int32)
        nblk = (K + blk - 1) // blk
        dma(0, R).start()

        def body(b, _):
            base = b * blk
            slot = jax.lax.rem(b, 2)

            @pl.when(b + 1 < nblk)
            def _():
                dma(jax.lax.rem(b + 1, 2), R + base + blk).start()

            dma(slot, R + base).wait()
            x = xbuf_ref[slot]
            gi = _dot(x.astype(jnp.bfloat16), wih) + bih
            h = out_ref[pl.ds(base, blk), :]
            gh = _dot(h.astype(jnp.bfloat16), whh) + bhh
            ir, iz, inew = gi[:, :d], gi[:, d:2 * d], gi[:, 2 * d:]
            hr, hz, hnn = gh[:, :d], gh[:, d:2 * d], gh[:, 2 * d:]
            r = jax.nn.sigmoid(ir + hr)
            z = jax.nn.sigmoid(iz + hz)
            cand = jnp.tanh(inew + r * hnn)
            hn = (1.0 - z) * cand + z * h
            rowid = base + jax.lax.broadcasted_iota(jnp.int32, (blk, 1), 0)
            out_ref[pl.ds(base, blk), :] = jnp.where(rowid < K, hn, h)
            return 0

        jax.lax.fori_loop(0, nblk, body, 0)
        return R + K

    jax.lax.fori_loop(0, nsteps, step, jnp.int32(0))


def _post_eopa_body(s_ref, h_ref, wneigh_t_ref, a0_ref, feat_ref,
                    g1_ref, b1_ref, wqv_t_ref, qvb_ref, wk_t_ref,
                    feat1_ref, qv_ref, k_ref):
    out0 = s_ref[...] + _dot(h_ref[...], wneigh_t_ref[...])
    a0 = a0_ref[...]
    out0 = jnp.where(out0 > 0, out0, a0 * out0)
    feat1 = jnp.concatenate([out0, feat_ref[...]], axis=1)
    feat1_ref[...] = feat1
    fb1 = _bn_cols(feat1, g1_ref[...], b1_ref[...])
    qv_ref[...] = _dot(fb1, wqv_t_ref[...]) + qvb_ref[...]
    k_ref[...] = _dot(fb1, wk_t_ref[...])


def _feat2_body(rst_ref, a1_ref, feat1_ref, feat2_ref):
    rst = rst_ref[...]
    a1 = a1_ref[...]
    out1 = jnp.where(rst > 0, rst, a1 * rst)
    feat2_ref[...] = jnp.concatenate([out1, feat1_ref[...]], axis=1)


def _bn2_body(feat2_ref, g2_ref, b2_ref, wu_t_ref, fb2_ref, fu_ref,
              mean_ref, rstd_ref):
    x = feat2_ref[...]
    m = jnp.mean(x, axis=0, keepdims=True)
    v = jnp.mean((x - m) ** 2, axis=0, keepdims=True)
    rs = jax.lax.rsqrt(v + 1e-5)
    mean_ref[...] = m
    rstd_ref[...] = rs
    fb2 = (x - m) * rs * g2_ref[...] + b2_ref[...]
    fb2_ref[...] = fb2
    fu_ref[...] = _dot(fb2, wu_t_ref[...])


def _fv_body(f2l_ref, mean_ref, rstd_ref, g2_ref, b2_ref, wv2_t_ref,
             bv2_ref, fv_ref):
    fb2l = ((f2l_ref[...] - mean_ref[...]) * rstd_ref[...] * g2_ref[...]
            + b2_ref[...])
    fv_ref[...] = _dot(fb2l, wv2_t_ref[...]) + bv2_ref[...]


def _read_a_body(fu_ref, seg_ref, fv_ref, we2_ref, e2_ref):
    nb = fu_ref.shape[0]
    nseg = fv_ref.shape[0]
    oh = (seg_ref[...] ==
          jax.lax.broadcasted_iota(jnp.int32, (nb, nseg), 1)).astype(
              jnp.float32)
    fvn = _dot(oh, fv_ref[...])
    sig = jax.nn.sigmoid(fu_ref[...] + fvn)
    e2_ref[...] = jnp.sum(sig * we2_ref[...], axis=1, keepdims=True)


def _read_b_body(e2_ref, mx_ref, seg_ref, fb2_ref, s_ref, den_ref):
    i = pl.program_id(0)

    @pl.when(i == 0)
    def _():
        s_ref[...] = jnp.zeros(s_ref.shape, s_ref.dtype)
        den_ref[...] = jnp.zeros(den_ref.shape, den_ref.dtype)

    nb = e2_ref.shape[0]
    nseg = s_ref.shape[0]
    ex = jnp.exp(e2_ref[...] - mx_ref[0])
    oh = (seg_ref[...] ==
          jax.lax.broadcasted_iota(jnp.int32, (nb, nseg), 1)).astype(
              jnp.float32)
    w = oh * ex
    dn = (((0,), (0,)), ((), ()))
    s_ref[...] += jax.lax.dot_general(
        w, fb2_ref[...], dn, preferred_element_type=jnp.float32)
    den_ref[...] += jax.lax.dot_general(
        w, jnp.ones((nb, den_ref.shape[1]), jnp.float32), dn,
        preferred_element_type=jnp.float32)


def _final_body(s_ref, den_ref, wout_t_ref, a2_ref, feat2l_ref, wrf_t_ref,
                rfb_ref, wsr_t_ref, dec_ref, logits_ref):
    den0 = den_ref[...][:, :1]
    srg_raw = jnp.where(den0 > 0, s_ref[...] / den0, 0.0)
    srg = _dot(srg_raw, wout_t_ref[...])
    a2 = a2_ref[...]
    srg = jnp.where(srg > 0, srg, a2 * srg)
    sr = jnp.concatenate([feat2l_ref[...], srg], axis=1)
    dec_ref[...] = _dot(sr, wrf_t_ref[...]) + rfb_ref[...]
    logits_ref[...] = _dot(sr, wsr_t_ref[...])


_SC_NW = 32
_SC_CH = 128


def _sc_gather_rows(table, idx):
    m = idx.shape[0]
    d = table.shape[1]
    per_w = m // _SC_NW
    nch = per_w // _SC_CH
    idxm = idx.reshape(m // _SC_CH, _SC_CH)
    mesh = plsc.VectorSubcoreMesh(core_axis_name="c", subcore_axis_name="s")

    nbuf = 4 if d <= 128 else 2

    @functools.partial(
        pl.kernel, mesh=mesh,
        out_type=jax.ShapeDtypeStruct((m, d), table.dtype),
        scratch_types=[
            pltpu.VMEM((nch, _SC_CH), jnp.int32),
            pltpu.VMEM((nbuf, _SC_CH, d), table.dtype),
            pltpu.SemaphoreType.DMA((nbuf,)),
            pltpu.SemaphoreType.DMA((nbuf,)),
        ],
    )
    def gather_k(table_hbm, idxm_hbm, out_hbm, idx_v, rows_v, gsem, wsem):
        wid = lax.axis_index("s") * 2 + lax.axis_index("c")
        base = wid * per_w
        pltpu.sync_copy(idxm_hbm.at[pl.ds(wid * nch, nch)], idx_v)
        for j in range(nbuf - 1):
            if j < nch:
                pltpu.async_copy(table_hbm.at[idx_v.at[j]], rows_v.at[j],
                                 gsem.at[j])

        def body(c, _):
            slot = lax.rem(c, nbuf)
            pltpu.make_async_copy(
                table_hbm.at[idx_v.at[c]], rows_v.at[slot],
                gsem.at[slot]).wait()
            pltpu.async_copy(
                rows_v.at[slot], out_hbm.at[pl.ds(base + c * _SC_CH, _SC_CH)],
                wsem.at[slot])

            @pl.when(c + nbuf - 1 < nch)
            def _():
                ns = lax.rem(c + nbuf - 1, nbuf)

                @pl.when(c >= 1)
                def _():
                    pltpu.make_async_copy(
                        rows_v.at[ns], out_hbm.at[pl.ds(base, _SC_CH)],
                        wsem.at[ns]).wait()

                pltpu.async_copy(
                    table_hbm.at[idx_v.at[c + nbuf - 1]], rows_v.at[ns],
                    gsem.at[ns])

            return 0

        lax.fori_loop(0, nch, body, 0, unroll=False)
        for c in range(max(nch - nbuf, 0), nch):
            pltpu.make_async_copy(
                rows_v.at[c % nbuf], out_hbm.at[pl.ds(base, _SC_CH)],
                wsem.at[c % nbuf]).wait()

    return gather_k(table, idxm)


def _sc_build_x(fb0, src_m, rank_m, dst_m, g_tab, nodepos, m_out, dump):
    nch_all, ch = src_m.shape
    m = nch_all * ch
    d = fb0.shape[1]
    per_w = m // _SC_NW
    nch = per_w // ch
    mesh = plsc.VectorSubcoreMesh(core_axis_name="c", subcore_axis_name="s")

    @functools.partial(
        pl.kernel, mesh=mesh,
        out_type=jax.ShapeDtypeStruct((m_out, d), fb0.dtype),
        scratch_types=[
            pltpu.VMEM((nch, ch), jnp.int32),
            pltpu.VMEM((nch, ch), jnp.int32),
            pltpu.VMEM((nch, ch), jnp.int32),
            pltpu.VMEM((2, ch), jnp.int32),
            pltpu.VMEM((2, ch), jnp.int32),
            pltpu.VMEM((2, ch), jnp.int32),
            pltpu.VMEM((2, ch, d), jnp.float32),
            pltpu.SemaphoreType.DMA((2,)),
            pltpu.SemaphoreType.DMA((2,)),
            pltpu.SemaphoreType.DMA((2,)),
            pltpu.SemaphoreType.DMA((2,)),
        ],
    )
    def build_k(fb_hbm, srcm_hbm, rankm_hbm, dstm_hbm, gt_hbm, np_hbm,
                out_hbm, src_v, rank_v, dst_v, gt_v, np_v, pos_v,
                rows_v, gsem, s1sem, s2sem, wsem):
        wid = lax.axis_index("s") * 2 + lax.axis_index("c")
        pltpu.sync_copy(srcm_hbm.at[pl.ds(wid * nch, nch)], src_v)
        pltpu.sync_copy(rankm_hbm.at[pl.ds(wid * nch, nch)], rank_v)
        pltpu.sync_copy(dstm_hbm.at[pl.ds(wid * nch, nch)], dst_v)
        pltpu.async_copy(fb_hbm.at[src_v.at[0]], rows_v.at[0], gsem.at[0])
        pltpu.async_copy(gt_hbm.at[rank_v.at[0]], gt_v.at[0], s1sem.at[0])
        pltpu.async_copy(np_hbm.at[dst_v.at[0]], np_v.at[0], s2sem.at[0])

        def body(c, _):
            slot = lax.rem(c, 2)
            nslot = lax.rem(c + 1, 2)

            @pl.when(c + 1 < nch)
            def _():
                @pl.when(c >= 1)
                def _():
                    pltpu.make_async_copy(rows_v.at[nslot],
                                          out_hbm.at[pos_v.at[nslot]],
                                          wsem.at[nslot]).wait()

                pltpu.async_copy(fb_hbm.at[src_v.at[c + 1]],
                                 rows_v.at[nslot], gsem.at[nslot])
                pltpu.async_copy(gt_hbm.at[rank_v.at[c + 1]],
                                 gt_v.at[nslot], s1sem.at[nslot])
                pltpu.async_copy(np_hbm.at[dst_v.at[c + 1]],
                                 np_v.at[nslot], s2sem.at[nslot])

            pltpu.make_async_copy(gt_hbm.at[rank_v.at[c]], gt_v.at[slot],
                                  s1sem.at[slot]).wait()
            pltpu.make_async_copy(np_hbm.at[dst_v.at[c]], np_v.at[slot],
                                  s2sem.at[slot]).wait()
            for s in (0, 1):
                @pl.when(slot == s)
                def _():
                    for i in range(ch // 16):
                        sl = pl.ds(i * 16, 16)
                        pos_v[s, sl] = jnp.minimum(
                            gt_v[s, sl] + np_v[s, sl], dump)

            pltpu.make_async_copy(fb_hbm.at[src_v.at[c]], rows_v.at[slot],
                                  gsem.at[slot]).wait()
            pltpu.async_copy(rows_v.at[slot], out_hbm.at[pos_v.at[slot]],
                             wsem.at[slot])
            return 0

        lax.fori_loop(0, nch, body, 0, unroll=False)
        for c in range(max(nch - 2, 0), nch):
            pltpu.make_async_copy(rows_v.at[c % 2],
                                  out_hbm.at[pos_v.at[c % 2]],
                                  wsem.at[c % 2]).wait()

    return build_k(fb0, src_m, rank_m, dst_m, g_tab, nodepos)


def _sc_scatter_add(rows, idx_m, zeros_init):
    m, dext = rows.shape
    nch_all, ch = idx_m.shape
    nseg = zeros_init.shape[0]
    per_w = m // _SC_NW
    nch = per_w // ch
    mesh = plsc.VectorSubcoreMesh(core_axis_name="c", subcore_axis_name="s")

    @functools.partial(
        pl.kernel, mesh=mesh,
        out_type=jax.ShapeDtypeStruct((2, nseg, dext), jnp.float32),
        scratch_types=[
            pltpu.VMEM((nch, ch), jnp.int32),
            pltpu.VMEM((2, ch, dext), jnp.float32),
            pltpu.VMEM_SHARED((nseg, dext), jnp.float32),
            pltpu.SemaphoreType.DMA((2,)),
            pltpu.SemaphoreType.DMA((2,)),
        ],
    )
    def scat_k(rows_hbm, idxm_hbm, zeros_hbm, out_hbm, idx_v, rows_v,
               acc_sh, lsem, ssem):
        cid = lax.axis_index("c")
        sid = lax.axis_index("s")
        wid = sid * 2 + cid

        @pl.when(sid == 0)
        def _():
            pltpu.sync_copy(zeros_hbm, acc_sh)

        plsc.subcore_barrier()
        pltpu.sync_copy(idxm_hbm.at[pl.ds(wid * nch, nch)], idx_v)
        base = wid * per_w
        pltpu.async_copy(rows_hbm.at[pl.ds(base, ch)], rows_v.at[0],
                         lsem.at[0])

        def body(c, _):
            slot = lax.rem(c, 2)
            nslot = lax.rem(c + 1, 2)

            @pl.when(c + 1 < nch)
            def _():
                @pl.when(c >= 1)
                def _():
                    pltpu.make_async_copy(
                        rows_v.at[nslot], acc_sh.at[idx_v.at[c - 1]],
                        ssem.at[nslot]).wait()

                pltpu.async_copy(
                    rows_hbm.at[pl.ds(base + (c + 1) * ch, ch)],
                    rows_v.at[nslot], lsem.at[nslot])

            pltpu.make_async_copy(
                rows_hbm.at[pl.ds(base, ch)], rows_v.at[slot],
                lsem.at[slot]).wait()
            pltpu.async_copy(rows_v.at[slot], acc_sh.at[idx_v.at[c]],
                             ssem.at[slot], add=True)
            return 0

        lax.fori_loop(0, nch, body, 0, unroll=False)
        for c in range(max(nch - 2, 0), nch):
            pltpu.make_async_copy(rows_v.at[c % 2],
                                  acc_sh.at[idx_v.at[c]],
                                  ssem.at[c % 2]).wait()
        plsc.subcore_barrier()

        @pl.when(sid == 0)
        def _():
            pltpu.sync_copy(acc_sh, out_hbm.at[cid])

    return scat_k(rows, idx_m, zeros_init)


def _edge_e_body(qvs_ref, kd_ref, w_ref, out_ref):
    d = kd_ref.shape[1]
    sig = jax.nn.sigmoid(qvs_ref[...][:, :d] + kd_ref[...])
    out_ref[...] = jnp.sum(sig * w_ref[...], axis=1, keepdims=True)


def _call(body, out_shapes, *args):
    return pl.pallas_call(
        body,
        out_shape=out_shapes,
    )(*args)


def _seg_softmax(e, seg, num):
    ex = jnp.exp(e - jnp.max(e))
    den = jax.ops.segment_sum(ex, seg, num_segments=num)
    return ex / den[seg]


def kernel(iid, edge_index_mg, edge_index_sg, segment_ids, last_nodes,
           emb_table, bn0_gamma, bn0_beta, gru_Wih, gru_Whh, gru_bih, gru_bhh,
           fc_self_W, fc_neigh_W, prelu0_a, bn1_gamma, bn1_beta,
           fc_q_W, fc_q_b, fc_k_W, fc_v_W, fc_e1_W, prelu1_a,
           bn2_gamma, bn2_beta, fc_u_W, fc_v2_W, fc_v2_b, fc_e2_W, fc_out_W,
           prelu2_a, fc_RF_W, fc_RF_b, fc_sr_W):
    n = iid.shape[0]
    d = emb_table.shape[1]
    b = last_nodes.shape[0]
    f32 = jnp.float32

    def row(x):
        return x.reshape(1, -1).astype(f32)

    feat = emb_table[iid]

    src, dst = edge_index_mg[0], edge_index_mg[1]
    e = src.shape[0]
    order = jnp.argsort(dst)
    src_s = src[order].astype(jnp.int32)
    dst_s = dst[order].astype(jnp.int32)
    counts = jnp.bincount(dst, length=n)
    ar = jnp.arange(e, dtype=jnp.int32)
    is_start = jnp.concatenate(
        [jnp.ones((1,), jnp.bool_), dst_s[1:] != dst_s[:-1]])
    seg_start = jax.lax.cummax(jnp.where(is_start, ar, 0))
    rank_s = ar - seg_start
    maxdeg = counts.max()

    fb0, S = _call(
        _pre_body,
        (jax.ShapeDtypeStruct((n, d), f32),
         jax.ShapeDtypeStruct((n, d), f32)),
        feat, row(bn0_gamma), row(bn0_beta), fc_self_W.T.astype(f32))

    blk = _GRU_BLK
    np_pad = ((n + blk - 1) // blk) * blk
    perm = jnp.argsort(-counts, stable=False)
    nodepos = jnp.zeros((n,), jnp.int32).at[perm].set(
        jnp.arange(n, dtype=jnp.int32))
    counts_p = jnp.zeros((np_pad,), jnp.int32).at[:n].set(counts[perm])
    countsf = counts_p.astype(f32).reshape(np_pad // 128, 128)

    hist = jnp.bincount(counts, length=e + 1)
    k_of_t = n - jnp.cumsum(hist)
    g_tab = jnp.concatenate(
        [jnp.zeros((1,), jnp.int32),
         jnp.cumsum(k_of_t).astype(jnp.int32)])
    m_e = ((e + 4095) // 4096) * 4096
    m_g = ((e + blk + 8 + 4095) // 4096) * 4096 + 4096

    def padm(x, fill):
        return jnp.full((m_e,), fill, jnp.int32).at[:e].set(x).reshape(
            m_e // _SC_CH, _SC_CH)

    x_stream = _sc_build_x(
        fb0, padm(src_s, 0), padm(rank_s, e + 1), padm(dst_s, 0),
        g_tab, nodepos, m_g, m_g - 1)

    nsteps = maxdeg.astype(jnp.int32).reshape(1)
    h_p = pl.pallas_call(
        _gru_mega_body,
        out_shape=jax.ShapeDtypeStruct((np_pad, d), f32),
        in_specs=[
            pl.BlockSpec(memory_space=pltpu.SMEM),
            pl.BlockSpec(memory_space=pltpu.MemorySpace.VMEM),
            pl.BlockSpec(memory_space=pltpu.MemorySpace.VMEM),
            pl.BlockSpec(memory_space=pltpu.MemorySpace.VMEM),
            pl.BlockSpec(memory_space=pltpu.MemorySpace.VMEM),
            pl.BlockSpec(memory_space=pltpu.MemorySpace.VMEM),
            pl.BlockSpec(memory_space=pltpu.MemorySpace.HBM),
        ],
        out_specs=pl.BlockSpec(memory_space=pltpu.MemorySpace.VMEM),
        scratch_shapes=[
            pltpu.VMEM((2, blk, d), f32),
            pltpu.SemaphoreType.DMA((2,)),
        ],
    )(nsteps, countsf, gru_Wih.T.astype(jnp.bfloat16), row(gru_bih),
      gru_Whh.T.astype(jnp.bfloat16), row(gru_bhh), x_stream)
    neigh = jnp.take(h_p, nodepos, axis=0)

    wqv_t = jnp.concatenate(
        [fc_q_W.T.astype(f32), fc_v_W.T.astype(f32)], axis=1)
    qvb = jnp.concatenate(
        [fc_q_b.astype(f32), jnp.zeros((d,), f32)]).reshape(1, 2 * d)
    feat1, qv, k = _call(
        _post_eopa_body,
        (jax.ShapeDtypeStruct((n, 2 * d), f32),
         jax.ShapeDtypeStruct((n, 2 * d), f32),
         jax.ShapeDtypeStruct((n, d), f32)),
        S, neigh, fc_neigh_W.T.astype(f32), row(prelu0_a), feat,
        row(bn1_gamma), row(bn1_beta), wqv_t, qvb, fc_k_W.T.astype(f32))

    src2 = edge_index_sg[0].astype(jnp.int32)
    dst2 = edge_index_sg[1].astype(jnp.int32)
    src2_p = jnp.zeros((m_e,), jnp.int32).at[:e].set(src2)
    dst2_p = jnp.zeros((m_e,), jnp.int32).at[:e].set(dst2)
    qvs = _sc_gather_rows(qv, src2_p)
    kd = _sc_gather_rows(k, dst2_p)

    eblk = 8192
    e_att = pl.pallas_call(
        _edge_e_body,
        grid=(m_e // eblk,),
        in_specs=[
            pl.BlockSpec((eblk, 2 * d), lambda i: (i, 0)),
            pl.BlockSpec((eblk, d), lambda i: (i, 0)),
            pl.BlockSpec((1, d), lambda i: (0, 0)),
        ],
        out_specs=pl.BlockSpec((eblk, 1), lambda i: (i, 0)),
        out_shape=jax.ShapeDtypeStruct((m_e, 1), f32),
    )(qvs, kd, fc_e1_W.astype(f32))
    ex_full = jnp.exp(e_att[:, 0] - jnp.max(e_att))
    wrows = qvs[:, d:] * ex_full[:, None]
    dst2_dump = jnp.full((m_e,), n, jnp.int32).at[:e].set(dst2)
    acc = _sc_scatter_add(
        wrows, dst2_dump.reshape(m_e // _SC_CH, _SC_CH),
        jnp.zeros((n + 8, d), f32))
    rst_u = (acc[0] + acc[1])[:n]
    den = jax.ops.segment_sum(ex_full[:e], dst2, num_segments=n)
    rst = jnp.where(den[:, None] > 0, rst_u / den[:, None], 0.0)

    feat2 = _call(
        _feat2_body,
        jax.ShapeDtypeStruct((n, 3 * d), f32),
        rst, row(prelu1_a), feat1)
    fb2, fu, mean2, rstd2 = _call(
        _bn2_body,
        (jax.ShapeDtypeStruct((n, 3 * d), f32),
         jax.ShapeDtypeStruct((n, d), f32),
         jax.ShapeDtypeStruct((1, 3 * d), f32),
         jax.ShapeDtypeStruct((1, 3 * d), f32)),
        feat2, row(bn2_gamma), row(bn2_beta), fc_u_W.T.astype(f32))

    feat2_last = feat2[last_nodes]
    fv = _call(
        _fv_body, jax.ShapeDtypeStruct((b, d), f32),
        feat2_last, mean2, rstd2, row(bn2_gamma), row(bn2_beta),
        fc_v2_W.T.astype(f32), row(fc_v2_b))

    segc = segment_ids.astype(jnp.int32).reshape(n, 1)
    rblk = 2000
    e2 = pl.pallas_call(
        _read_a_body,
        grid=(n // rblk,),
        in_specs=[
            pl.BlockSpec((rblk, d), lambda i: (i, 0)),
            pl.BlockSpec((rblk, 1), lambda i: (i, 0)),
            pl.BlockSpec((b, d), lambda i: (0, 0)),
            pl.BlockSpec((1, d), lambda i: (0, 0)),
        ],
        out_specs=pl.BlockSpec((rblk, 1), lambda i: (i, 0)),
        out_shape=jax.ShapeDtypeStruct((n, 1), f32),
    )(fu, segc, fv, fc_e2_W.astype(f32))

    mx = jnp.max(e2).reshape(1)
    s_acc, den = pl.pallas_call(
        _read_b_body,
        grid=(n // rblk,),
        in_specs=[
            pl.BlockSpec((rblk, 1), lambda i: (i, 0)),
            pl.BlockSpec(memory_space=pltpu.SMEM),
            pl.BlockSpec((rblk, 1), lambda i: (i, 0)),
            pl.BlockSpec((rblk, 3 * d), lambda i: (i, 0)),
        ],
        out_specs=(pl.BlockSpec((b, 3 * d), lambda i: (0, 0)),
                   pl.BlockSpec((b, 8), lambda i: (0, 0))),
        out_shape=(jax.ShapeDtypeStruct((b, 3 * d), f32),
                   jax.ShapeDtypeStruct((b, 8), f32)),
    )(e2, mx, segc, fb2)

    dec, logits = _call(
        _final_body,
        (jax.ShapeDtypeStruct((b, fc_RF_W.shape[0]), f32),
         jax.ShapeDtypeStruct((b, fc_sr_W.shape[0]), f32)),
        s_acc, den, fc_out_W.T.astype(f32), row(prelu2_a), feat2_last,
        fc_RF_W.T.astype(f32), row(fc_RF_b), fc_sr_W.T.astype(f32))

    return (dec, logits)


# speedup vs baseline: 10.0943x; 1.0850x over previous
import functools

import jax
import jax.numpy as jnp
from jax import lax
from jax.experimental import pallas as pl
from jax.experimental.pallas import tpu as pltpu
from jax.experimental.pallas import tpu_sc as plsc


def _bn_cols(x, g, b):
    m = jnp.mean(x, axis=0, keepdims=True)
    v = jnp.mean((x - m) ** 2, axis=0, keepdims=True)
    return (x - m) * jax.lax.rsqrt(v + 1e-5) * g + b


def _dot(a, b):
    return jnp.dot(a, b, preferred_element_type=jnp.float32)


def _pre_body(feat_ref, g_ref, b_ref, wself_t_ref, fb_ref, s_ref):
    fb = _bn_cols(feat_ref[...], g_ref[...], b_ref[...])
    fb_ref[...] = fb
    s_ref[...] = _dot(fb, wself_t_ref[...])


_GRU_BLK = 2048


def _gru_mega_body(nsteps_ref, countsf_ref, wih_t_ref, bih_ref, whh_t_ref,
                   bhh_ref, xs_ref, out_ref, xbuf_ref, sem_ref):
    blk = _GRU_BLK
    d = out_ref.shape[1]
    out_ref[...] = jnp.zeros(out_ref.shape, out_ref.dtype)
    nsteps = nsteps_ref[0]
    cf = countsf_ref[...]
    wih = wih_t_ref[...]
    whh = whh_t_ref[...]
    bih = bih_ref[...]
    bhh = bhh_ref[...]

    def dma(slot, start):
        return pltpu.make_async_copy(
            xs_ref.at[pl.ds(start, blk)], xbuf_ref.at[slot], sem_ref.at[slot])

    def step(t, R):
        tf = t.astype(jnp.float32)
        K = jnp.sum((cf > tf).astype(jnp.float32)).astype(jnp.int32)
        nblk = (K + blk - 1) // blk
        dma(0, R).start()

        def body(b, _):
            base = b * blk
            slot = jax.lax.rem(b, 2)

            @pl.when(b + 1 < nblk)
            def _():
                dma(jax.lax.rem(b + 1, 2), R + base + blk).start()

            dma(slot, R + base).wait()
            x = xbuf_ref[slot]
            gi = _dot(x.astype(jnp.bfloat16), wih) + bih
            h = out_ref[pl.ds(base, blk), :]
            gh = _dot(h.astype(jnp.bfloat16), whh) + bhh
            ir, iz, inew = gi[:, :d], gi[:, d:2 * d], gi[:, 2 * d:]
            hr, hz, hnn = gh[:, :d], gh[:, d:2 * d], gh[:, 2 * d:]
            r = jax.nn.sigmoid(ir + hr)
            z = jax.nn.sigmoid(iz + hz)
            cand = jnp.tanh(inew + r * hnn)
            hn = (1.0 - z) * cand + z * h
            rowid = base + jax.lax.broadcasted_iota(jnp.int32, (blk, 1), 0)
            out_ref[pl.ds(base, blk), :] = jnp.where(rowid < K, hn, h)
            return 0

        jax.lax.fori_loop(0, nblk, body, 0)
        return R + K

    jax.lax.fori_loop(0, nsteps, step, jnp.int32(0))


def _post_eopa_body(s_ref, h_ref, wneigh_t_ref, a0_ref, feat_ref,
                    g1_ref, b1_ref, wqv_t_ref, qvb_ref, wk_t_ref,
                    feat1_ref, qv_ref, k_ref):
    out0 = s_ref[...] + _dot(h_ref[...], wneigh_t_ref[...])
    a0 = a0_ref[...]
    out0 = jnp.where(out0 > 0, out0, a0 * out0)
    feat1 = jnp.concatenate([out0, feat_ref[...]], axis=1)
    feat1_ref[...] = feat1
    fb1 = _bn_cols(feat1, g1_ref[...], b1_ref[...])
    qv_ref[...] = _dot(fb1, wqv_t_ref[...]) + qvb_ref[...]
    k_ref[...] = _dot(fb1, wk_t_ref[...])


def _feat2_body(rst_ref, a1_ref, feat1_ref, feat2_ref):
    rst = rst_ref[...]
    a1 = a1_ref[...]
    out1 = jnp.where(rst > 0, rst, a1 * rst)
    feat2_ref[...] = jnp.concatenate([out1, feat1_ref[...]], axis=1)


def _bn2_body(feat2_ref, g2_ref, b2_ref, wu_t_ref, fb2_ref, fu_ref,
              mean_ref, rstd_ref):
    x = feat2_ref[...]
    m = jnp.mean(x, axis=0, keepdims=True)
    v = jnp.mean((x - m) ** 2, axis=0, keepdims=True)
    rs = jax.lax.rsqrt(v + 1e-5)
    mean_ref[...] = m
    rstd_ref[...] = rs
    fb2 = (x - m) * rs * g2_ref[...] + b2_ref[...]
    fb2_ref[...] = fb2
    fu_ref[...] = _dot(fb2, wu_t_ref[...])


def _fv_body(f2l_ref, mean_ref, rstd_ref, g2_ref, b2_ref, wv2_t_ref,
             bv2_ref, fv_ref):
    fb2l = ((f2l_ref[...] - mean_ref[...]) * rstd_ref[...] * g2_ref[...]
            + b2_ref[...])
    fv_ref[...] = _dot(fb2l, wv2_t_ref[...]) + bv2_ref[...]


def _read_a_body(fu_ref, seg_ref, fv_ref, we2_ref, e2_ref):
    nb = fu_ref.shape[0]
    nseg = fv_ref.shape[0]
    oh = (seg_ref[...] ==
          jax.lax.broadcasted_iota(jnp.int32, (nb, nseg), 1)).astype(
              jnp.float32)
    fvn = _dot(oh, fv_ref[...])
    sig = jax.nn.sigmoid(fu_ref[...] + fvn)
    e2_ref[...] = jnp.sum(sig * we2_ref[...], axis=1, keepdims=True)


def _read_b_body(e2_ref, mx_ref, seg_ref, fb2_ref, s_ref, den_ref):
    i = pl.program_id(0)

    @pl.when(i == 0)
    def _():
        s_ref[...] = jnp.zeros(s_ref.shape, s_ref.dtype)
        den_ref[...] = jnp.zeros(den_ref.shape, den_ref.dtype)

    nb = e2_ref.shape[0]
    nseg = s_ref.shape[0]
    ex = jnp.exp(e2_ref[...] - mx_ref[0])
    oh = (seg_ref[...] ==
          jax.lax.broadcasted_iota(jnp.int32, (nb, nseg), 1)).astype(
              jnp.float32)
    w = oh * ex
    dn = (((0,), (0,)), ((), ()))
    s_ref[...] += jax.lax.dot_general(
        w, fb2_ref[...], dn, preferred_element_type=jnp.float32)
    den_ref[...] += jax.lax.dot_general(
        w, jnp.ones((nb, den_ref.shape[1]), jnp.float32), dn,
        preferred_element_type=jnp.float32)


def _final_body(s_ref, den_ref, wout_t_ref, a2_ref, feat2l_ref, wrf_t_ref,
                rfb_ref, wsr_t_ref, dec_ref, logits_ref):
    den0 = den_ref[...][:, :1]
    srg_raw = jnp.where(den0 > 0, s_ref[...] / den0, 0.0)
    srg = _dot(srg_raw, wout_t_ref[...])
    a2 = a2_ref[...]
    srg = jnp.where(srg > 0, srg, a2 * srg)
    sr = jnp.concatenate([feat2l_ref[...], srg], axis=1)
    dec_ref[...] = _dot(sr, wrf_t_ref[...]) + rfb_ref[...]
    logits_ref[...] = _dot(sr, wsr_t_ref[...])


_SC_NW = 32
_SC_CH = 128


def _sc_gather_rows(table, idx):
    m = idx.shape[0]
    d = table.shape[1]
    per_w = m // _SC_NW
    nch = per_w // _SC_CH
    idxm = idx.reshape(m // _SC_CH, _SC_CH)
    mesh = plsc.VectorSubcoreMesh(core_axis_name="c", subcore_axis_name="s")

    nbuf = 4 if d <= 128 else 2

    @functools.partial(
        pl.kernel, mesh=mesh,
        out_type=jax.ShapeDtypeStruct((m, d), table.dtype),
        scratch_types=[
            pltpu.VMEM((nch, _SC_CH), jnp.int32),
            pltpu.VMEM((nbuf, _SC_CH, d), table.dtype),
            pltpu.SemaphoreType.DMA((nbuf,)),
            pltpu.SemaphoreType.DMA((nbuf,)),
        ],
    )
    def gather_k(table_hbm, idxm_hbm, out_hbm, idx_v, rows_v, gsem, wsem):
        wid = lax.axis_index("s") * 2 + lax.axis_index("c")
        base = wid * per_w
        pltpu.sync_copy(idxm_hbm.at[pl.ds(wid * nch, nch)], idx_v)
        for j in range(nbuf - 1):
            if j < nch:
                pltpu.async_copy(table_hbm.at[idx_v.at[j]], rows_v.at[j],
                                 gsem.at[j])

        def body(c, _):
            slot = lax.rem(c, nbuf)
            pltpu.make_async_copy(
                table_hbm.at[idx_v.at[c]], rows_v.at[slot],
                gsem.at[slot]).wait()
            pltpu.async_copy(
                rows_v.at[slot], out_hbm.at[pl.ds(base + c * _SC_CH, _SC_CH)],
                wsem.at[slot])

            @pl.when(c + nbuf - 1 < nch)
            def _():
                ns = lax.rem(c + nbuf - 1, nbuf)

                @pl.when(c >= 1)
                def _():
                    pltpu.make_async_copy(
                        rows_v.at[ns], out_hbm.at[pl.ds(base, _SC_CH)],
                        wsem.at[ns]).wait()

                pltpu.async_copy(
                    table_hbm.at[idx_v.at[c + nbuf - 1]], rows_v.at[ns],
                    gsem.at[ns])

            return 0

        lax.fori_loop(0, nch, body, 0, unroll=False)
        for c in range(max(nch - nbuf, 0), nch):
            pltpu.make_async_copy(
                rows_v.at[c % nbuf], out_hbm.at[pl.ds(base, _SC_CH)],
                wsem.at[c % nbuf]).wait()

    return gather_k(table, idxm)


def _sc_build_x(fb0, src_m, rank_m, dst_m, g_tab, nodepos, m_out, dump):
    nch_all, ch = src_m.shape
    m = nch_all * ch
    d = fb0.shape[1]
    per_w = m // _SC_NW
    nch = per_w // ch
    mesh = plsc.VectorSubcoreMesh(core_axis_name="c", subcore_axis_name="s")

    @functools.partial(
        pl.kernel, mesh=mesh,
        out_type=jax.ShapeDtypeStruct((m_out, d), fb0.dtype),
        scratch_types=[
            pltpu.VMEM((nch, ch), jnp.int32),
            pltpu.VMEM((nch, ch), jnp.int32),
            pltpu.VMEM((nch, ch), jnp.int32),
            pltpu.VMEM((2, ch), jnp.int32),
            pltpu.VMEM((2, ch), jnp.int32),
            pltpu.VMEM((2, ch), jnp.int32),
            pltpu.VMEM((2, ch, d), jnp.float32),
            pltpu.SemaphoreType.DMA((2,)),
            pltpu.SemaphoreType.DMA((2,)),
            pltpu.SemaphoreType.DMA((2,)),
            pltpu.SemaphoreType.DMA((2,)),
        ],
    )
    def build_k(fb_hbm, srcm_hbm, rankm_hbm, dstm_hbm, gt_hbm, np_hbm,
                out_hbm, src_v, rank_v, dst_v, gt_v, np_v, pos_v,
                rows_v, gsem, s1sem, s2sem, wsem):
        wid = lax.axis_index("s") * 2 + lax.axis_index("c")
        pltpu.sync_copy(srcm_hbm.at[pl.ds(wid * nch, nch)], src_v)
        pltpu.sync_copy(rankm_hbm.at[pl.ds(wid * nch, nch)], rank_v)
        pltpu.sync_copy(dstm_hbm.at[pl.ds(wid * nch, nch)], dst_v)
        pltpu.async_copy(fb_hbm.at[src_v.at[0]], rows_v.at[0], gsem.at[0])
        pltpu.async_copy(gt_hbm.at[rank_v.at[0]], gt_v.at[0], s1sem.at[0])
        pltpu.async_copy(np_hbm.at[dst_v.at[0]], np_v.at[0], s2sem.at[0])

        def body(c, _):
            slot = lax.rem(c, 2)
            nslot = lax.rem(c + 1, 2)

            @pl.when(c + 1 < nch)
            def _():
                @pl.when(c >= 1)
                def _():
                    pltpu.make_async_copy(rows_v.at[nslot],
                                          out_hbm.at[pos_v.at[nslot]],
                                          wsem.at[nslot]).wait()

                pltpu.async_copy(fb_hbm.at[src_v.at[c + 1]],
                                 rows_v.at[nslot], gsem.at[nslot])
                pltpu.async_copy(gt_hbm.at[rank_v.at[c + 1]],
                                 gt_v.at[nslot], s1sem.at[nslot])
                pltpu.async_copy(np_hbm.at[dst_v.at[c + 1]],
                                 np_v.at[nslot], s2sem.at[nslot])

            pltpu.make_async_copy(gt_hbm.at[rank_v.at[c]], gt_v.at[slot],
                                  s1sem.at[slot]).wait()
            pltpu.make_async_copy(np_hbm.at[dst_v.at[c]], np_v.at[slot],
                                  s2sem.at[slot]).wait()
            for s in (0, 1):
                @pl.when(slot == s)
                def _():
                    for i in range(ch // 16):
                        sl = pl.ds(i * 16, 16)
                        pos_v[s, sl] = jnp.minimum(
                            gt_v[s, sl] + np_v[s, sl], dump)

            pltpu.make_async_copy(fb_hbm.at[src_v.at[c]], rows_v.at[slot],
                                  gsem.at[slot]).wait()
            pltpu.async_copy(rows_v.at[slot], out_hbm.at[pos_v.at[slot]],
                             wsem.at[slot])
            return 0

        lax.fori_loop(0, nch, body, 0, unroll=False)
        for c in range(max(nch - 2, 0), nch):
            pltpu.make_async_copy(rows_v.at[c % 2],
                                  out_hbm.at[pos_v.at[c % 2]],
                                  wsem.at[c % 2]).wait()

    return build_k(fb0, src_m, rank_m, dst_m, g_tab, nodepos)


def _sc_gather_two(table_a, idx_a, table_b, idx_b):
    m = idx_a.shape[0]
    da = table_a.shape[1]
    db = table_b.shape[1]
    per_w = m // _SC_NW
    nch = per_w // _SC_CH
    idxm_a = idx_a.reshape(m // _SC_CH, _SC_CH)
    idxm_b = idx_b.reshape(m // _SC_CH, _SC_CH)
    mesh = plsc.VectorSubcoreMesh(core_axis_name="c", subcore_axis_name="s")

    @functools.partial(
        pl.kernel, mesh=mesh,
        out_type=(jax.ShapeDtypeStruct((m, da), table_a.dtype),
                  jax.ShapeDtypeStruct((m, db), table_b.dtype)),
        scratch_types=[
            pltpu.VMEM((nch, _SC_CH), jnp.int32),
            pltpu.VMEM((nch, _SC_CH), jnp.int32),
            pltpu.VMEM((2, _SC_CH, da), table_a.dtype),
            pltpu.VMEM((2, _SC_CH, db), table_b.dtype),
            pltpu.SemaphoreType.DMA((2,)),
            pltpu.SemaphoreType.DMA((2,)),
            pltpu.SemaphoreType.DMA((2,)),
            pltpu.SemaphoreType.DMA((2,)),
        ],
    )
    def gather2_k(ta_hbm, ia_hbm, tb_hbm, ib_hbm, oa_hbm, ob_hbm,
                  ia_v, ib_v, ra_v, rb_v, gasem, gbsem, wasem, wbsem):
        wid = lax.axis_index("s") * 2 + lax.axis_index("c")
        base = wid * per_w
        pltpu.sync_copy(ia_hbm.at[pl.ds(wid * nch, nch)], ia_v)
        pltpu.sync_copy(ib_hbm.at[pl.ds(wid * nch, nch)], ib_v)
        pltpu.async_copy(ta_hbm.at[ia_v.at[0]], ra_v.at[0], gasem.at[0])
        pltpu.async_copy(tb_hbm.at[ib_v.at[0]], rb_v.at[0], gbsem.at[0])

        def body(c, _):
            slot = lax.rem(c, 2)
            nslot = lax.rem(c + 1, 2)

            @pl.when(c + 1 < nch)
            def _():
                @pl.when(c >= 1)
                def _():
                    pltpu.make_async_copy(
                        ra_v.at[nslot], oa_hbm.at[pl.ds(base, _SC_CH)],
                        wasem.at[nslot]).wait()
                    pltpu.make_async_copy(
                        rb_v.at[nslot], ob_hbm.at[pl.ds(base, _SC_CH)],
                        wbsem.at[nslot]).wait()

                pltpu.async_copy(ta_hbm.at[ia_v.at[c + 1]], ra_v.at[nslot],
                                 gasem.at[nslot])
                pltpu.async_copy(tb_hbm.at[ib_v.at[c + 1]], rb_v.at[nslot],
                                 gbsem.at[nslot])

            pltpu.make_async_copy(ta_hbm.at[ia_v.at[c]], ra_v.at[slot],
                                  gasem.at[slot]).wait()
            pltpu.async_copy(
                ra_v.at[slot], oa_hbm.at[pl.ds(base + c * _SC_CH, _SC_CH)],
                wasem.at[slot])
            pltpu.make_async_copy(tb_hbm.at[ib_v.at[c]], rb_v.at[slot],
                                  gbsem.at[slot]).wait()
            pltpu.async_copy(
                rb_v.at[slot], ob_hbm.at[pl.ds(base + c * _SC_CH, _SC_CH)],
                wbsem.at[slot])
            return 0

        lax.fori_loop(0, nch, body, 0, unroll=False)
        for c in range(max(nch - 2, 0), nch):
            pltpu.make_async_copy(ra_v.at[c % 2],
                                  oa_hbm.at[pl.ds(base, _SC_CH)],
                                  wasem.at[c % 2]).wait()
            pltpu.make_async_copy(rb_v.at[c % 2],
                                  ob_hbm.at[pl.ds(base, _SC_CH)],
                                  wbsem.at[c % 2]).wait()

    return gather2_k(table_a, idxm_a, table_b, idxm_b)


def _sc_scatter_add(rows, idx_m, zeros_init):
    m, dext = rows.shape
    nch_all, ch = idx_m.shape
    nseg = zeros_init.shape[0]
    per_w = m // _SC_NW
    nch = per_w // ch
    mesh = plsc.VectorSubcoreMesh(core_axis_name="c", subcore_axis_name="s")

    @functools.partial(
        pl.kernel, mesh=mesh,
        out_type=jax.ShapeDtypeStruct((2, nseg, dext), jnp.float32),
        scratch_types=[
            pltpu.VMEM((nch, ch), jnp.int32),
            pltpu.VMEM((2, ch, dext), jnp.float32),
            pltpu.VMEM_SHARED((nseg, dext), jnp.float32),
            pltpu.SemaphoreType.DMA((2,)),
            pltpu.SemaphoreType.DMA((2,)),
        ],
    )
    def scat_k(rows_hbm, idxm_hbm, zeros_hbm, out_hbm, idx_v, rows_v,
               acc_sh, lsem, ssem):
        cid = lax.axis_index("c")
        sid = lax.axis_index("s")
        wid = sid * 2 + cid

        @pl.when(sid == 0)
        def _():
            pltpu.sync_copy(zeros_hbm, acc_sh)

        plsc.subcore_barrier()
        pltpu.sync_copy(idxm_hbm.at[pl.ds(wid * nch, nch)], idx_v)
        base = wid * per_w
        pltpu.async_copy(rows_hbm.at[pl.ds(base, ch)], rows_v.at[0],
                         lsem.at[0])

        def body(c, _):
            slot = lax.rem(c, 2)
            nslot = lax.rem(c + 1, 2)

            @pl.when(c + 1 < nch)
            def _():
                @pl.when(c >= 1)
                def _():
                    pltpu.make_async_copy(
                        rows_v.at[nslot], acc_sh.at[idx_v.at[c - 1]],
                        ssem.at[nslot]).wait()

                pltpu.async_copy(
                    rows_hbm.at[pl.ds(base + (c + 1) * ch, ch)],
                    rows_v.at[nslot], lsem.at[nslot])

            pltpu.make_async_copy(
                rows_hbm.at[pl.ds(base, ch)], rows_v.at[slot],
                lsem.at[slot]).wait()
            pltpu.async_copy(rows_v.at[slot], acc_sh.at[idx_v.at[c]],
                             ssem.at[slot], add=True)
            return 0

        lax.fori_loop(0, nch, body, 0, unroll=False)
        for c in range(max(nch - 2, 0), nch):
            pltpu.make_async_copy(rows_v.at[c % 2],
                                  acc_sh.at[idx_v.at[c]],
                                  ssem.at[c % 2]).wait()
        plsc.subcore_barrier()

        @pl.when(sid == 0)
        def _():
            pltpu.sync_copy(acc_sh, out_hbm.at[cid])

    return scat_k(rows, idx_m, zeros_init)


def _edge_e_body(qvs_ref, kd_ref, w_ref, out_ref):
    d = kd_ref.shape[1]
    sig = jax.nn.sigmoid(qvs_ref[...][:, :d] + kd_ref[...])
    out_ref[...] = jnp.sum(sig * w_ref[...], axis=1, keepdims=True)


def _call(body, out_shapes, *args):
    return pl.pallas_call(
        body,
        out_shape=out_shapes,
    )(*args)


def _seg_softmax(e, seg, num):
    ex = jnp.exp(e - jnp.max(e))
    den = jax.ops.segment_sum(ex, seg, num_segments=num)
    return ex / den[seg]


def kernel(iid, edge_index_mg, edge_index_sg, segment_ids, last_nodes,
           emb_table, bn0_gamma, bn0_beta, gru_Wih, gru_Whh, gru_bih, gru_bhh,
           fc_self_W, fc_neigh_W, prelu0_a, bn1_gamma, bn1_beta,
           fc_q_W, fc_q_b, fc_k_W, fc_v_W, fc_e1_W, prelu1_a,
           bn2_gamma, bn2_beta, fc_u_W, fc_v2_W, fc_v2_b, fc_e2_W, fc_out_W,
           prelu2_a, fc_RF_W, fc_RF_b, fc_sr_W):
    n = iid.shape[0]
    d = emb_table.shape[1]
    b = last_nodes.shape[0]
    f32 = jnp.float32

    def row(x):
        return x.reshape(1, -1).astype(f32)

    feat = emb_table[iid]

    src, dst = edge_index_mg[0], edge_index_mg[1]
    e = src.shape[0]
    order = jnp.argsort(dst)
    src_s = src[order].astype(jnp.int32)
    dst_s = dst[order].astype(jnp.int32)
    counts = jnp.bincount(dst, length=n)
    ar = jnp.arange(e, dtype=jnp.int32)
    is_start = jnp.concatenate(
        [jnp.ones((1,), jnp.bool_), dst_s[1:] != dst_s[:-1]])
    seg_start = jax.lax.cummax(jnp.where(is_start, ar, 0))
    rank_s = ar - seg_start
    maxdeg = counts.max()

    fb0, S = _call(
        _pre_body,
        (jax.ShapeDtypeStruct((n, d), f32),
         jax.ShapeDtypeStruct((n, d), f32)),
        feat, row(bn0_gamma), row(bn0_beta), fc_self_W.T.astype(f32))

    blk = _GRU_BLK
    np_pad = ((n + blk - 1) // blk) * blk
    perm = jnp.argsort(-counts, stable=False)
    nodepos = jnp.zeros((n,), jnp.int32).at[perm].set(
        jnp.arange(n, dtype=jnp.int32))
    counts_p = jnp.zeros((np_pad,), jnp.int32).at[:n].set(counts[perm])
    countsf = counts_p.astype(f32).reshape(np_pad // 128, 128)

    hist = jnp.bincount(counts, length=e + 1)
    k_of_t = n - jnp.cumsum(hist)
    g_tab = jnp.concatenate(
        [jnp.zeros((1,), jnp.int32),
         jnp.cumsum(k_of_t).astype(jnp.int32)])
    m_e = ((e + 4095) // 4096) * 4096
    m_g = ((e + blk + 8 + 4095) // 4096) * 4096 + 4096

    def padm(x, fill):
        return jnp.full((m_e,), fill, jnp.int32).at[:e].set(x).reshape(
            m_e // _SC_CH, _SC_CH)

    x_stream = _sc_build_x(
        fb0, padm(src_s, 0), padm(rank_s, e + 1), padm(dst_s, 0),
        g_tab, nodepos, m_g, m_g - 1)

    nsteps = maxdeg.astype(jnp.int32).reshape(1)
    h_p = pl.pallas_call(
        _gru_mega_body,
        out_shape=jax.ShapeDtypeStruct((np_pad, d), f32),
        in_specs=[
            pl.BlockSpec(memory_space=pltpu.SMEM),
            pl.BlockSpec(memory_space=pltpu.MemorySpace.VMEM),
            pl.BlockSpec(memory_space=pltpu.MemorySpace.VMEM),
            pl.BlockSpec(memory_space=pltpu.MemorySpace.VMEM),
            pl.BlockSpec(memory_space=pltpu.MemorySpace.VMEM),
            pl.BlockSpec(memory_space=pltpu.MemorySpace.VMEM),
            pl.BlockSpec(memory_space=pltpu.MemorySpace.HBM),
        ],
        out_specs=pl.BlockSpec(memory_space=pltpu.MemorySpace.VMEM),
        scratch_shapes=[
            pltpu.VMEM((2, blk, d), f32),
            pltpu.SemaphoreType.DMA((2,)),
        ],
    )(nsteps, countsf, gru_Wih.T.astype(jnp.bfloat16), row(gru_bih),
      gru_Whh.T.astype(jnp.bfloat16), row(gru_bhh), x_stream)
    neigh = jnp.take(h_p, nodepos, axis=0)

    wqv_t = jnp.concatenate(
        [fc_q_W.T.astype(f32), fc_v_W.T.astype(f32)], axis=1)
    qvb = jnp.concatenate(
        [fc_q_b.astype(f32), jnp.zeros((d,), f32)]).reshape(1, 2 * d)
    feat1, qv, k = _call(
        _post_eopa_body,
        (jax.ShapeDtypeStruct((n, 2 * d), f32),
         jax.ShapeDtypeStruct((n, 2 * d), f32),
         jax.ShapeDtypeStruct((n, d), f32)),
        S, neigh, fc_neigh_W.T.astype(f32), row(prelu0_a), feat,
        row(bn1_gamma), row(bn1_beta), wqv_t, qvb, fc_k_W.T.astype(f32))

    src2 = edge_index_sg[0].astype(jnp.int32)
    dst2 = edge_index_sg[1].astype(jnp.int32)
    src2_p = jnp.zeros((m_e,), jnp.int32).at[:e].set(src2)
    dst2_p = jnp.zeros((m_e,), jnp.int32).at[:e].set(dst2)
    qvs, kd = _sc_gather_two(qv, src2_p, k, dst2_p)

    eblk = 8192
    e_att = pl.pallas_call(
        _edge_e_body,
        grid=(m_e // eblk,),
        in_specs=[
            pl.BlockSpec((eblk, 2 * d), lambda i: (i, 0)),
            pl.BlockSpec((eblk, d), lambda i: (i, 0)),
            pl.BlockSpec((1, d), lambda i: (0, 0)),
        ],
        out_specs=pl.BlockSpec((eblk, 1), lambda i: (i, 0)),
        out_shape=jax.ShapeDtypeStruct((m_e, 1), f32),
    )(qvs, kd, fc_e1_W.astype(f32))
    ex_full = jnp.exp(e_att[:, 0] - jnp.max(e_att))
    wrows = qvs[:, d:] * ex_full[:, None]
    dst2_dump = jnp.full((m_e,), n, jnp.int32).at[:e].set(dst2)
    acc = _sc_scatter_add(
        wrows, dst2_dump.reshape(m_e // _SC_CH, _SC_CH),
        jnp.zeros((n + 8, d), f32))
    rst_u = (acc[0] + acc[1])[:n]
    den = jax.ops.segment_sum(ex_full[:e], dst2, num_segments=n)
    rst = jnp.where(den[:, None] > 0, rst_u / den[:, None], 0.0)

    feat2 = _call(
        _feat2_body,
        jax.ShapeDtypeStruct((n, 3 * d), f32),
        rst, row(prelu1_a), feat1)
    fb2, fu, mean2, rstd2 = _call(
        _bn2_body,
        (jax.ShapeDtypeStruct((n, 3 * d), f32),
         jax.ShapeDtypeStruct((n, d), f32),
         jax.ShapeDtypeStruct((1, 3 * d), f32),
         jax.ShapeDtypeStruct((1, 3 * d), f32)),
        feat2, row(bn2_gamma), row(bn2_beta), fc_u_W.T.astype(f32))

    feat2_last = feat2[last_nodes]
    fv = _call(
        _fv_body, jax.ShapeDtypeStruct((b, d), f32),
        feat2_last, mean2, rstd2, row(bn2_gamma), row(bn2_beta),
        fc_v2_W.T.astype(f32), row(fc_v2_b))

    segc = segment_ids.astype(jnp.int32).reshape(n, 1)
    rblk = 2000
    e2 = pl.pallas_call(
        _read_a_body,
        grid=(n // rblk,),
        in_specs=[
            pl.BlockSpec((rblk, d), lambda i: (i, 0)),
            pl.BlockSpec((rblk, 1), lambda i: (i, 0)),
            pl.BlockSpec((b, d), lambda i: (0, 0)),
            pl.BlockSpec((1, d), lambda i: (0, 0)),
        ],
        out_specs=pl.BlockSpec((rblk, 1), lambda i: (i, 0)),
        out_shape=jax.ShapeDtypeStruct((n, 1), f32),
    )(fu, segc, fv, fc_e2_W.astype(f32))

    mx = jnp.max(e2).reshape(1)
    s_acc, den = pl.pallas_call(
        _read_b_body,
        grid=(n // rblk,),
        in_specs=[
            pl.BlockSpec((rblk, 1), lambda i: (i, 0)),
            pl.BlockSpec(memory_space=pltpu.SMEM),
            pl.BlockSpec((rblk, 1), lambda i: (i, 0)),
            pl.BlockSpec((rblk, 3 * d), lambda i: (i, 0)),
        ],
        out_specs=(pl.BlockSpec((b, 3 * d), lambda i: (0, 0)),
                   pl.BlockSpec((b, 8), lambda i: (0, 0))),
        out_shape=(jax.ShapeDtypeStruct((b, 3 * d), f32),
                   jax.ShapeDtypeStruct((b, 8), f32)),
    )(e2, mx, segc, fb2)

    dec, logits = _call(
        _final_body,
        (jax.ShapeDtypeStruct((b, fc_RF_W.shape[0]), f32),
         jax.ShapeDtypeStruct((b, fc_sr_W.shape[0]), f32)),
        s_acc, den, fc_out_W.T.astype(f32), row(prelu2_a), feat2_last,
        fc_RF_W.T.astype(f32), row(fc_RF_b), fc_sr_W.T.astype(f32))

    return (dec, logits)
